# Initial kernel scaffold; baseline (speedup 1.0000x reference)
import jax
import jax.numpy as jnp
from jax.experimental import pallas as pl


def kernel(x, edge_index, batch, W1, a_src1, a_dst1, b1, W2, a_src2, a_dst2, b2, Wg, bg, Wf1, bf1, Wf2, bf2, Wo, bo):
    raise NotImplementedError("write your pallas kernel here")


# speedup vs baseline: 2.4620x; 2.4620x over previous
import functools

import jax
import jax.numpy as jnp
from jax.experimental import pallas as pl
from jax.experimental.pallas import tpu as pltpu


def _mm_att_kernel(x_ref, w_ref, a_ref, h_ref, s_ref, *, heads, out_ch):
    h = jnp.dot(x_ref[...], w_ref[...], preferred_element_type=jnp.float32)
    h_ref[...] = h
    cols = []
    for k in range(2):
        for hd in range(heads):
            sl = slice(hd * out_ch, (hd + 1) * out_ch)
            prod = h[:, sl] * a_ref[k, sl][None, :]
            cols.append(jnp.sum(prod, axis=1, keepdims=True))
    s = jnp.concatenate(cols, axis=1)
    s_ref[...] = jnp.pad(s, ((0, 0), (0, 128 - 2 * heads)))


def _matmul_att(x, w, a2, block_rows, heads, out_ch):
    n, k = x.shape
    k2, m = w.shape
    grid = n // block_rows
    return pl.pallas_call(
        functools.partial(_mm_att_kernel, heads=heads, out_ch=out_ch),
        grid=(grid,),
        in_specs=[
            pl.BlockSpec((block_rows, k), lambda i: (i, 0)),
            pl.BlockSpec((k, m), lambda i: (0, 0)),
            pl.BlockSpec((2, m), lambda i: (0, 0)),
        ],
        out_specs=[
            pl.BlockSpec((block_rows, m), lambda i: (i, 0)),
            pl.BlockSpec((block_rows, 128), lambda i: (i, 0)),
        ],
        out_shape=[
            jax.ShapeDtypeStruct((n, m), jnp.float32),
            jax.ShapeDtypeStruct((n, 128), jnp.float32),
        ],
    )(x, w, a2)


def _mlp_kernel(p_ref, wg_ref, bg_ref, w1_ref, b1_ref, w2_ref, b2_ref,
                wo_ref, bo_ref, o_ref):
    h = jnp.maximum(jnp.dot(p_ref[...], wg_ref[...],
                            preferred_element_type=jnp.float32)
                    + bg_ref[...], 0.0)
    h = jnp.maximum(jnp.dot(h, w1_ref[...],
                            preferred_element_type=jnp.float32)
                    + b1_ref[...], 0.0)
    h = jnp.maximum(jnp.dot(h, w2_ref[...],
                            preferred_element_type=jnp.float32)
                    + b2_ref[...], 0.0)
    o_ref[...] = jnp.dot(h, wo_ref[...],
                         preferred_element_type=jnp.float32) + bo_ref[...]


def _mlp(pooled, Wg, bg, Wf1, bf1, Wf2, bf2, Wo, bo):
    G = pooled.shape[0]
    Wo_pad = jnp.zeros((16, 128), jnp.float32).at[:, 0].set(Wo[:, 0])
    bo_pad = jnp.zeros((1, 128), jnp.float32).at[0, 0].set(bo[0])
    out = pl.pallas_call(
        _mlp_kernel,
        out_shape=jax.ShapeDtypeStruct((G, 128), jnp.float32),
    )(pooled, Wg, bg.reshape(1, -1), Wf1, bf1.reshape(1, -1),
      Wf2, bf2.reshape(1, -1), Wo_pad, bo_pad)
    return out[:, :1]


def _gat_layer(x, src, dst, W, a_src, a_dst, b, heads, out_ch):
    N = x.shape[0]
    a2 = jnp.stack([a_src.reshape(-1), a_dst.reshape(-1)])
    h, s = _matmul_att(x, W, a2, 1000, heads, out_ch)
    s_src = s[:, :heads]
    s_dst = s[:, heads:2 * heads]

    e = jax.nn.leaky_relu(s_src[src] + s_dst[dst], 0.2)
    ee = jnp.exp(e - jnp.max(e))
    denom = jax.ops.segment_sum(ee, dst, num_segments=N)
    alpha = ee / (denom[dst] + 1e-16)
    msg = (h[src].reshape(-1, heads, out_ch) * alpha[..., None])
    out = jax.ops.segment_sum(msg.reshape(-1, heads * out_ch), dst,
                              num_segments=N)
    return out + b


def kernel(x, edge_index, batch, W1, a_src1, a_dst1, b1, W2, a_src2,
           a_dst2, b2, Wg, bg, Wf1, bf1, Wf2, bf2, Wo, bo):
    N, F = x.shape
    G = 256
    loop = jnp.arange(N, dtype=edge_index.dtype)
    src = jnp.concatenate([edge_index[0], loop])
    dst = jnp.concatenate([edge_index[1], loop])

    x1 = jax.nn.elu(_gat_layer(x, src, dst, W1, a_src1, a_dst1, b1,
                               a_src1.shape[0], F))
    x2 = jax.nn.relu(_gat_layer(x1, src, dst, W2, a_src2, a_dst2, b2,
                                1, W2.shape[1]))

    pooled = jax.ops.segment_max(x2, batch, num_segments=G)
    counts = jax.ops.segment_sum(jnp.ones((N,), jnp.float32), batch,
                                 num_segments=G)
    pooled = jnp.where(counts[:, None] > 0, pooled, 0.0)
    return _mlp(pooled, Wg, bg, Wf1, bf1, Wf2, bf2, Wo, bo)


# speedup vs baseline: 13.0741x; 5.3103x over previous
import functools

import jax
import jax.numpy as jnp
from jax import lax
from jax.experimental import pallas as pl
from jax.experimental.pallas import tpu as pltpu
from jax.experimental.pallas import tpu_sc as plsc

N = 10000
NP = 10112
ZR = NP // 16
E = 160000
ETOT = E + N
K = 128
EP = 172032
NC, NS = 2, 16
U1 = EP // (NC * NS * K)
U4 = EP // (NS * K)

_f32 = jnp.float32
_i32 = jnp.int32

_MESH = plsc.VectorSubcoreMesh(core_axis_name="c", subcore_axis_name="s")


def _mm_att_kernel(x_ref, w_ref, a_ref, h_ref, s_ref, *, heads, out_ch):
    h = jnp.dot(x_ref[...], w_ref[...], preferred_element_type=jnp.float32)
    h_ref[...] = h
    cols = []
    for k in range(2):
        for hd in range(heads):
            sl = slice(hd * out_ch, (hd + 1) * out_ch)
            prod = h[:, sl] * a_ref[k, sl][None, :]
            cols.append(jnp.sum(prod, axis=1, keepdims=True))
    s = jnp.concatenate(cols, axis=1)
    s_ref[...] = jnp.pad(s, ((0, 0), (0, 128 - 2 * heads)))


def _matmul_att(x, w, a2, block_rows, heads, out_ch):
    n, k = x.shape
    _, m = w.shape
    grid = n // block_rows
    return pl.pallas_call(
        functools.partial(_mm_att_kernel, heads=heads, out_ch=out_ch),
        grid=(grid,),
        in_specs=[
            pl.BlockSpec((block_rows, k), lambda i: (i, 0)),
            pl.BlockSpec((k, m), lambda i: (0, 0)),
            pl.BlockSpec((2, m), lambda i: (0, 0)),
        ],
        out_specs=[
            pl.BlockSpec((block_rows, m), lambda i: (i, 0)),
            pl.BlockSpec((block_rows, 128), lambda i: (i, 0)),
        ],
        out_shape=[
            jax.ShapeDtypeStruct((n, m), jnp.float32),
            jax.ShapeDtypeStruct((n, 128), jnp.float32),
        ],
    )(x, w, a2)


def _mlp_kernel(p_ref, wg_ref, bg_ref, w1_ref, b1_ref, w2_ref, b2_ref,
                wo_ref, bo_ref, o_ref):
    h = jnp.maximum(jnp.dot(p_ref[...], wg_ref[...],
                            preferred_element_type=jnp.float32)
                    + bg_ref[...], 0.0)
    h = jnp.maximum(jnp.dot(h, w1_ref[...],
                            preferred_element_type=jnp.float32)
                    + b1_ref[...], 0.0)
    h = jnp.maximum(jnp.dot(h, w2_ref[...],
                            preferred_element_type=jnp.float32)
                    + b2_ref[...], 0.0)
    o_ref[...] = jnp.dot(h, wo_ref[...],
                         preferred_element_type=jnp.float32) + bo_ref[...]


def _mlp(pooled, Wg, bg, Wf1, bf1, Wf2, bf2, Wo, bo):
    G = pooled.shape[0]
    Wo_pad = jnp.zeros((16, 128), jnp.float32).at[:, 0].set(Wo[:, 0])
    bo_pad = jnp.zeros((1, 128), jnp.float32).at[0, 0].set(bo[0])
    out = pl.pallas_call(
        _mlp_kernel,
        out_shape=jax.ShapeDtypeStruct((G, 128), jnp.float32),
    )(pooled, Wg, bg.reshape(1, -1), Wf1, bf1.reshape(1, -1),
      Wf2, bf2.reshape(1, -1), Wo_pad, bo_pad)
    return out[:, :1]


NPF = NP * 4


def _p1_body(svs, svd, srcp, dstp, cvec, *rest, heads):
    ee_outs = rest[:heads]
    den_out = rest[heads]
    scr = rest[heads + 1:]
    svs_v, svd_v, den_v, idx_s, idx_d, cv_b = scr[:6]
    ee_bufs = scr[6:6 + heads]
    c = lax.axis_index("c")
    s = lax.axis_index("s")
    tid = c * NS + s

    pltpu.sync_copy(svs, svs_v)
    pltpu.sync_copy(svd, svd_v)
    pltpu.sync_copy(cvec, cv_b)
    cv = cv_b[...]

    def zrow(i, _):
        den_v[pl.ds(16 * i, 16)] = jnp.zeros((16,), _f32)
        return 0
    lax.fori_loop(0, NPF // 16, zrow, 0)

    def step(st, _):
        gbase = (tid * U1 + st) * K
        pltpu.sync_copy(srcp.at[pl.ds(gbase, K)], idx_s)
        pltpu.sync_copy(dstp.at[pl.ds(gbase, K)], idx_d)
        for g in range(K // 16):
            sl = pl.ds(16 * g, 16)
            i_s4 = idx_s[sl] * 4
            i_d4 = idx_d[sl] * 4
            for h in range(heads):
                ss = plsc.load_gather(svs_v, [i_s4 + h])
                sd = plsc.load_gather(svd_v, [i_d4 + h])
                t = ss + sd
                e = jnp.where(t >= 0.0, t, 0.2 * t)
                ee = jnp.exp(e - cv)
                ee_bufs[h][sl] = ee
                plsc.addupdate_scatter(den_v, [i_d4 + h], ee)
        for h in range(heads):
            pltpu.sync_copy(ee_bufs[h], ee_outs[h].at[pl.ds(gbase, K)])
        return 0
    lax.fori_loop(0, U1, step, 0)
    pltpu.sync_copy(den_v, den_out.at[tid])


def _make_p1(heads):
    return pl.kernel(
        functools.partial(_p1_body, heads=heads),
        out_type=[jax.ShapeDtypeStruct((EP,), _f32)] * heads
        + [jax.ShapeDtypeStruct((NC * NS, NPF), _f32)],
        mesh=_MESH,
        scratch_types=[
            pltpu.VMEM((NPF,), _f32), pltpu.VMEM((NPF,), _f32),
            pltpu.VMEM((NPF,), _f32),
            pltpu.VMEM((K,), _i32), pltpu.VMEM((K,), _i32),
            pltpu.VMEM((16,), _f32),
        ] + [pltpu.VMEM((K,), _f32)] * heads,
        compiler_params=pltpu.CompilerParams(needs_layout_passes=False),
    )


_p1_h3 = _make_p1(3)
_p1_h1 = _make_p1(1)


def _p4_body(*refs, n_j, mult, steps, split_cores):
    h_all = refs[0]
    ees = refs[1:1 + n_j]
    srcp, dstp, agg_out = refs[1 + n_j:4 + n_j]
    idx_s, idx_d, idx_a, rows, ee_b, acc, sem = refs[4 + n_j:]
    c = lax.axis_index("c")
    s = lax.axis_index("s")

    for j in range(n_j):
        chunk = 2 * j + c if mult > 1 else c * 0
        def zrow(i, _):
            for t in range(8):
                rows[i, pl.ds(16 * t, 16)] = jnp.zeros((16,), _f32)
            return 0
        lax.fori_loop(0, K, zrow, 0)
        for r in range(4):
            pltpu.sync_copy(rows, acc.at[pl.ds(s * ZR + r * K, K)])
        pltpu.sync_copy(rows.at[pl.ds(0, ZR - 4 * K)],
                        acc.at[pl.ds(s * ZR + 4 * K, ZR - 4 * K)])
        plsc.subcore_barrier()

        def step(st, _):
            if split_cores:
                gbase = ((c * NS + s) * steps + st) * K
            else:
                gbase = (s * steps + st) * K
            pltpu.sync_copy(srcp.at[pl.ds(gbase, K)], idx_s)
            pltpu.sync_copy(dstp.at[pl.ds(gbase, K)], idx_d)
            pltpu.sync_copy(ees[j].at[pl.ds(gbase, K)], ee_b)
            if mult > 1:
                for k8 in range(K // 16):
                    sl = pl.ds(16 * k8, 16)
                    idx_a[sl] = idx_s[sl] * mult + chunk
                gather_idx = idx_a
            else:
                gather_idx = idx_s
            pltpu.async_copy(h_all.at[gather_idx], rows, sem).wait()

            def wg(g, _):
                wv = ee_b[pl.ds(16 * g, 16)]
                for lane in range(16):
                    w = wv[lane]
                    k = g * 16 + lane
                    for t in range(8):
                        sl = pl.ds(16 * t, 16)
                        rows[k, sl] = rows[k, sl] * w
                return 0
            lax.fori_loop(0, K // 16, wg, 0)
            pltpu.sync_copy(rows, acc.at[idx_d], add=True)
            return 0
        lax.fori_loop(0, steps, step, 0)
        plsc.subcore_barrier()
        if mult > 1:
            pltpu.sync_copy(acc.at[pl.ds(s * ZR, ZR)],
                            agg_out.at[chunk, pl.ds(s * ZR, ZR)])
        else:
            pltpu.sync_copy(acc.at[pl.ds(s * ZR, ZR)],
                            agg_out.at[c, pl.ds(s * ZR, ZR)])


def _make_p4(n_j, mult, steps, split_cores, out_shape):
    return pl.kernel(
        functools.partial(_p4_body, n_j=n_j, mult=mult, steps=steps,
                          split_cores=split_cores),
        out_type=jax.ShapeDtypeStruct(out_shape, _f32),
        mesh=_MESH,
        scratch_types=[
            pltpu.VMEM((K,), _i32), pltpu.VMEM((K,), _i32),
            pltpu.VMEM((K,), _i32),
            pltpu.VMEM((K, 128), _f32), pltpu.VMEM((K,), _f32),
            pltpu.VMEM_SHARED((NP, 128), _f32),
            pltpu.SemaphoreType.DMA,
        ],
        compiler_params=pltpu.CompilerParams(needs_layout_passes=False),
    )


_p4_l1 = _make_p4(3, 6, U4, False, (6, NP, 128))
_p4_l2 = _make_p4(1, 1, U1, True, (NC, NP, 128))


def _edge_layer(h_all, s_scores, heads, srcp, dstp, p1, p4):
    s_src = s_scores[:, :heads]
    s_dst = s_scores[:, heads:2 * heads]
    C = jnp.max(s_src) + jnp.max(s_dst)
    cvec = jnp.full((16,), C, _f32)
    svs = jnp.pad(s_src, ((0, NP - N), (0, 4 - heads))).reshape(NPF)
    svd = jnp.pad(s_dst, ((0, NP - N), (0, 4 - heads))).reshape(NPF)
    outs = p1(svs, svd, srcp, dstp, cvec)
    ees, den_part = outs[:heads], outs[heads]
    den = den_part.sum(axis=0).reshape(NP, 4)[:N, :heads]
    agg = p4(h_all, *ees, srcp, dstp)
    return agg, den


def kernel(x, edge_index, batch, W1, a_src1, a_dst1, b1, W2, a_src2,
           a_dst2, b2, Wg, bg, Wf1, bf1, Wf2, bf2, Wo, bo):
    G = 256
    H1 = a_src1.shape[0]
    loop = jnp.arange(N, dtype=edge_index.dtype)
    padi = jnp.full((EP - ETOT,), N, edge_index.dtype)
    srcp = jnp.concatenate([edge_index[0], loop, padi])
    dstp = jnp.concatenate([edge_index[1], loop, padi])

    a2_1 = jnp.stack([a_src1.reshape(-1), a_dst1.reshape(-1)])
    h1, s1 = _matmul_att(x, W1, a2_1, 1000, H1, x.shape[1])
    h_all1 = jnp.pad(h1, ((0, NP - N), (0, 0))).reshape(NP * 6, 128)
    agg1, den1s = _edge_layer(h_all1, s1, H1, srcp, dstp, _p1_h3, _p4_l1)
    out1 = agg1[:, :N].transpose(1, 0, 2).reshape(N, H1, 256)
    x1 = jax.nn.elu(out1 / (den1s[..., None] + 1e-16)
                    + b1.reshape(H1, 256)).reshape(N, H1 * 256)

    a2_2 = jnp.stack([a_src2.reshape(-1), a_dst2.reshape(-1)])
    h2, s2 = _matmul_att(x1, W2, a2_2, 1000, 1, W2.shape[1])
    h2p = jnp.pad(h2, ((0, NP - N), (0, 0)))
    agg2, den2s = _edge_layer(h2p, s2, 1, srcp, dstp, _p1_h1, _p4_l2)
    x2 = jax.nn.relu((agg2[0] + agg2[1])[:N] / (den2s + 1e-16) + b2)

    pooled = jax.ops.segment_max(x2, batch, num_segments=G)
    counts = jax.ops.segment_sum(jnp.ones((N,), jnp.float32), batch,
                                 num_segments=G)
    pooled = jnp.where(counts[:, None] > 0, pooled, 0.0)
    return _mlp(pooled, Wg, bg, Wf1, bf1, Wf2, bf2, Wo, bo)


# speedup vs baseline: 18.2963x; 1.3994x over previous
import functools

import jax
import jax.numpy as jnp
from jax import lax
from jax.experimental import pallas as pl
from jax.experimental.pallas import tpu as pltpu
from jax.experimental.pallas import tpu_sc as plsc

N = 10000
NP = 10112
ZR = NP // 16
E = 160000
ETOT = E + N
K = 128
EP = 172032
NC, NS = 2, 16
U1 = EP // (NC * NS * K)
U4 = EP // (NS * K)

_f32 = jnp.float32
_i32 = jnp.int32

_MESH = plsc.VectorSubcoreMesh(core_axis_name="c", subcore_axis_name="s")


def _mm_att_kernel(x_ref, w_ref, a_ref, h_ref, s_ref, *, heads, out_ch):
    h = jnp.dot(x_ref[...], w_ref[...], preferred_element_type=jnp.float32)
    h_ref[...] = h
    cols = []
    for k in range(2):
        for hd in range(heads):
            sl = slice(hd * out_ch, (hd + 1) * out_ch)
            prod = h[:, sl] * a_ref[k, sl][None, :]
            cols.append(jnp.sum(prod, axis=1, keepdims=True))
    s = jnp.concatenate(cols, axis=1)
    s_ref[...] = jnp.pad(s, ((0, 0), (0, 128 - 2 * heads)))


def _matmul_att(x, w, a2, block_rows, heads, out_ch):
    n, k = x.shape
    _, m = w.shape
    grid = n // block_rows
    return pl.pallas_call(
        functools.partial(_mm_att_kernel, heads=heads, out_ch=out_ch),
        grid=(grid,),
        in_specs=[
            pl.BlockSpec((block_rows, k), lambda i: (i, 0)),
            pl.BlockSpec((k, m), lambda i: (0, 0)),
            pl.BlockSpec((2, m), lambda i: (0, 0)),
        ],
        out_specs=[
            pl.BlockSpec((block_rows, m), lambda i: (i, 0)),
            pl.BlockSpec((block_rows, 128), lambda i: (i, 0)),
        ],
        out_shape=[
            jax.ShapeDtypeStruct((n, m), jnp.float32),
            jax.ShapeDtypeStruct((n, 128), jnp.float32),
        ],
    )(x, w, a2)


def _mlp_kernel(p_ref, wg_ref, bg_ref, w1_ref, b1_ref, w2_ref, b2_ref,
                wo_ref, bo_ref, o_ref):
    h = jnp.maximum(jnp.dot(p_ref[...], wg_ref[...],
                            preferred_element_type=jnp.float32)
                    + bg_ref[...], 0.0)
    h = jnp.maximum(jnp.dot(h, w1_ref[...],
                            preferred_element_type=jnp.float32)
                    + b1_ref[...], 0.0)
    h = jnp.maximum(jnp.dot(h, w2_ref[...],
                            preferred_element_type=jnp.float32)
                    + b2_ref[...], 0.0)
    o_ref[...] = jnp.dot(h, wo_ref[...],
                         preferred_element_type=jnp.float32) + bo_ref[...]


def _mlp(pooled, Wg, bg, Wf1, bf1, Wf2, bf2, Wo, bo):
    G = pooled.shape[0]
    Wo_pad = jnp.zeros((16, 128), jnp.float32).at[:, 0].set(Wo[:, 0])
    bo_pad = jnp.zeros((1, 128), jnp.float32).at[0, 0].set(bo[0])
    out = pl.pallas_call(
        _mlp_kernel,
        out_shape=jax.ShapeDtypeStruct((G, 128), jnp.float32),
    )(pooled, Wg, bg.reshape(1, -1), Wf1, bf1.reshape(1, -1),
      Wf2, bf2.reshape(1, -1), Wo_pad, bo_pad)
    return out[:, :1]


NPF = NP * 4


def _p1_body(svs, svd, srcp, dstp, cvec, *rest, heads):
    ee_outs = rest[:heads]
    den_out = rest[heads]
    scr = rest[heads + 1:]
    svs_v, svd_v, den_v, idx_s, idx_d, cv_b = scr[:6]
    ee_bufs = scr[6:6 + heads]
    c = lax.axis_index("c")
    s = lax.axis_index("s")
    tid = c * NS + s

    pltpu.sync_copy(svs, svs_v)
    pltpu.sync_copy(svd, svd_v)
    pltpu.sync_copy(cvec, cv_b)
    cv = cv_b[...]

    def zrow(i, _):
        den_v[pl.ds(16 * i, 16)] = jnp.zeros((16,), _f32)
        return 0
    lax.fori_loop(0, NPF // 16, zrow, 0)

    def step(st, _):
        gbase = (tid * U1 + st) * K
        pltpu.sync_copy(srcp.at[pl.ds(gbase, K)], idx_s)
        pltpu.sync_copy(dstp.at[pl.ds(gbase, K)], idx_d)
        for g in range(K // 16):
            sl = pl.ds(16 * g, 16)
            i_s4 = idx_s[sl] * 4
            i_d4 = idx_d[sl] * 4
            for h in range(heads):
                ss = plsc.load_gather(svs_v, [i_s4 + h])
                sd = plsc.load_gather(svd_v, [i_d4 + h])
                t = ss + sd
                e = jnp.where(t >= 0.0, t, 0.2 * t)
                ee = jnp.exp(e - cv)
                ee_bufs[h][sl] = ee
                plsc.addupdate_scatter(den_v, [i_d4 + h], ee)
        for h in range(heads):
            pltpu.sync_copy(ee_bufs[h], ee_outs[h].at[pl.ds(gbase, K)])
        return 0
    lax.fori_loop(0, U1, step, 0)
    pltpu.sync_copy(den_v, den_out.at[tid])


def _make_p1(heads):
    return pl.kernel(
        functools.partial(_p1_body, heads=heads),
        out_type=[jax.ShapeDtypeStruct((EP,), _f32)] * heads
        + [jax.ShapeDtypeStruct((NC * NS, NPF), _f32)],
        mesh=_MESH,
        scratch_types=[
            pltpu.VMEM((NPF,), _f32), pltpu.VMEM((NPF,), _f32),
            pltpu.VMEM((NPF,), _f32),
            pltpu.VMEM((K,), _i32), pltpu.VMEM((K,), _i32),
            pltpu.VMEM((16,), _f32),
        ] + [pltpu.VMEM((K,), _f32)] * heads,
        compiler_params=pltpu.CompilerParams(needs_layout_passes=False),
    )


_p1_h3 = _make_p1(3)
_p1_h1 = _make_p1(1)


def _p4_body(*refs, n_j, steps, split_cores):
    h_all = refs[0]
    pks = refs[1:1 + n_j]
    agg_out = refs[1 + n_j]
    scr = refs[2 + n_j:]
    pkb = scr[0:2]
    idx_a = scr[2:4]
    idx_d = scr[4:6]
    rows = scr[6:8]
    gsem = scr[8:10]
    acc = scr[10]
    c = lax.axis_index("c")
    s = lax.axis_index("s")

    def gb(st):
        if split_cores:
            return ((c * NS + s) * steps + st) * K
        return (s * steps + st) * K

    for j in range(n_j):
        chunk = 2 * j + c if n_j > 1 else c * 0
        pk = pks[j]

        def zrow(i, _):
            for t in range(8):
                rows[0][i, pl.ds(16 * t, 16)] = jnp.zeros((16,), _f32)
            return 0
        lax.fori_loop(0, K, zrow, 0)
        for r in range(4):
            pltpu.sync_copy(rows[0], acc.at[pl.ds(s * ZR + r * K, K)])
        pltpu.sync_copy(rows[0].at[pl.ds(0, ZR - 4 * K)],
                        acc.at[pl.ds(s * ZR + 4 * K, ZR - 4 * K)])
        plsc.subcore_barrier()

        def prep(b, st):
            pltpu.sync_copy(pk.at[:, pl.ds(gb(st), K)], pkb[b])
            for g16 in range(8):
                sl = pl.ds(16 * g16, 16)
                idx_a[b][sl] = pkb[b][0, sl] + chunk
                idx_d[b][sl] = pkb[b][1, sl]

        def gather(b):
            return pltpu.async_copy(h_all.at[idx_a[b]], rows[b], gsem[b])

        def compute_scatter(b):
            def wg(g16, _):
                wv = plsc.bitcast(pkb[b][2, pl.ds(16 * g16, 16)], _f32)
                for lane in range(16):
                    w = wv[lane]
                    k = g16 * 16 + lane
                    for t in range(8):
                        sl = pl.ds(16 * t, 16)
                        rows[b][k, sl] = rows[b][k, sl] * w
                return 0
            lax.fori_loop(0, 8, wg, 0)
            pltpu.sync_copy(rows[b], acc.at[idx_d[b]], add=True)

        prep(0, 0)
        g0 = gather(0)
        prep(1, 1)

        def body2(i, _):
            pltpu.make_async_copy(h_all.at[idx_a[0]], rows[0],
                                  gsem[0]).wait()
            gather(1)
            compute_scatter(0)

            @pl.when(i < steps // 2 - 1)
            def _():
                prep(0, 2 * i + 2)
            pltpu.make_async_copy(h_all.at[idx_a[1]], rows[1],
                                  gsem[1]).wait()

            @pl.when(i < steps // 2 - 1)
            def _():
                gather(0)
            compute_scatter(1)

            @pl.when(i < steps // 2 - 1)
            def _():
                prep(1, 2 * i + 3)
            return 0
        lax.fori_loop(0, steps // 2, body2, 0)
        plsc.subcore_barrier()
        if n_j > 1:
            pltpu.sync_copy(acc.at[pl.ds(s * ZR, ZR)],
                            agg_out.at[chunk, pl.ds(s * ZR, ZR)])
        else:
            pltpu.sync_copy(acc.at[pl.ds(s * ZR, ZR)],
                            agg_out.at[c, pl.ds(s * ZR, ZR)])


def _make_p4(n_j, steps, split_cores, out_shape):
    return pl.kernel(
        functools.partial(_p4_body, n_j=n_j, steps=steps,
                          split_cores=split_cores),
        out_type=jax.ShapeDtypeStruct(out_shape, _f32),
        mesh=_MESH,
        scratch_types=[
            pltpu.VMEM((3, K), _i32), pltpu.VMEM((3, K), _i32),
            pltpu.VMEM((K,), _i32), pltpu.VMEM((K,), _i32),
            pltpu.VMEM((K,), _i32), pltpu.VMEM((K,), _i32),
            pltpu.VMEM((K, 128), _f32), pltpu.VMEM((K, 128), _f32),
            pltpu.SemaphoreType.DMA, pltpu.SemaphoreType.DMA,
            pltpu.VMEM_SHARED((NP, 128), _f32),
        ],
        compiler_params=pltpu.CompilerParams(needs_layout_passes=False),
    )


_p4_l1 = _make_p4(3, U4, False, (6, NP, 128))
_p4_l2 = _make_p4(1, U1, True, (NC, NP, 128))


def _edge_layer(h_all, s_scores, heads, srcp, dstp, p1, p4):
    s_src = s_scores[:, :heads]
    s_dst = s_scores[:, heads:2 * heads]
    C = jnp.max(s_src) + jnp.max(s_dst)
    cvec = jnp.full((16,), C, _f32)
    svs = jnp.pad(s_src, ((0, NP - N), (0, 4 - heads))).reshape(NPF)
    svd = jnp.pad(s_dst, ((0, NP - N), (0, 4 - heads))).reshape(NPF)
    outs = p1(svs, svd, srcp, dstp, cvec)
    ees, den_part = outs[:heads], outs[heads]
    den = den_part.sum(axis=0).reshape(NP, 4)[:N, :heads]
    mult = 6 if heads > 1 else 1
    src_m = srcp * mult
    pks = [jnp.stack([src_m, dstp, lax.bitcast_convert_type(ee, _i32)])
           for ee in ees]
    agg = p4(h_all, *pks)
    return agg, den


def kernel(x, edge_index, batch, W1, a_src1, a_dst1, b1, W2, a_src2,
           a_dst2, b2, Wg, bg, Wf1, bf1, Wf2, bf2, Wo, bo):
    G = 256
    H1 = a_src1.shape[0]
    loop = jnp.arange(N, dtype=edge_index.dtype)
    padi = jnp.full((EP - ETOT,), N, edge_index.dtype)
    srcp = jnp.concatenate([edge_index[0], loop, padi])
    dstp = jnp.concatenate([edge_index[1], loop, padi])

    a2_1 = jnp.stack([a_src1.reshape(-1), a_dst1.reshape(-1)])
    h1, s1 = _matmul_att(x, W1, a2_1, 1000, H1, x.shape[1])
    h_all1 = jnp.pad(h1, ((0, NP - N), (0, 0))).reshape(NP * 6, 128)
    agg1, den1s = _edge_layer(h_all1, s1, H1, srcp, dstp, _p1_h3, _p4_l1)
    out1 = agg1[:, :N].transpose(1, 0, 2).reshape(N, H1, 256)
    x1 = jax.nn.elu(out1 / (den1s[..., None] + 1e-16)
                    + b1.reshape(H1, 256)).reshape(N, H1 * 256)

    a2_2 = jnp.stack([a_src2.reshape(-1), a_dst2.reshape(-1)])
    h2, s2 = _matmul_att(x1, W2, a2_2, 1000, 1, W2.shape[1])
    h2p = jnp.pad(h2, ((0, NP - N), (0, 0)))
    agg2, den2s = _edge_layer(h2p, s2, 1, srcp, dstp, _p1_h1, _p4_l2)
    x2 = jax.nn.relu((agg2[0] + agg2[1])[:N] / (den2s + 1e-16) + b2)

    pooled = jax.ops.segment_max(x2, batch, num_segments=G)
    counts = jax.ops.segment_sum(jnp.ones((N,), jnp.float32), batch,
                                 num_segments=G)
    pooled = jnp.where(counts[:, None] > 0, pooled, 0.0)
    return _mlp(pooled, Wg, bg, Wf1, bf1, Wf2, bf2, Wo, bo)


# speedup vs baseline: 19.2178x; 1.0504x over previous
import functools

import jax
import jax.numpy as jnp
from jax import lax
from jax.experimental import pallas as pl
from jax.experimental.pallas import tpu as pltpu
from jax.experimental.pallas import tpu_sc as plsc

N = 10000
NP = 10112
ZR = NP // 16
E = 160000
ETOT = E + N
K = 128
EP = 172032
NC, NS = 2, 16
U1 = EP // (NC * NS * K)
U4 = EP // (NS * K)

_f32 = jnp.float32
_i32 = jnp.int32

_MESH = plsc.VectorSubcoreMesh(core_axis_name="c", subcore_axis_name="s")


def _mm_att_kernel(x_ref, w_ref, a_ref, h_ref, s_ref, *, heads, out_ch):
    h = jnp.dot(x_ref[...], w_ref[...], preferred_element_type=jnp.float32)
    h_ref[...] = h
    cols = []
    for k in range(2):
        for hd in range(heads):
            sl = slice(hd * out_ch, (hd + 1) * out_ch)
            prod = h[:, sl] * a_ref[k, sl][None, :]
            cols.append(jnp.sum(prod, axis=1, keepdims=True))
    s = jnp.concatenate(cols, axis=1)
    s_ref[...] = jnp.pad(s, ((0, 0), (0, 128 - 2 * heads)))


def _matmul_att(x, w, a2, block_rows, heads, out_ch):
    n, k = x.shape
    _, m = w.shape
    grid = n // block_rows
    return pl.pallas_call(
        functools.partial(_mm_att_kernel, heads=heads, out_ch=out_ch),
        grid=(grid,),
        in_specs=[
            pl.BlockSpec((block_rows, k), lambda i: (i, 0)),
            pl.BlockSpec((k, m), lambda i: (0, 0)),
            pl.BlockSpec((2, m), lambda i: (0, 0)),
        ],
        out_specs=[
            pl.BlockSpec((block_rows, m), lambda i: (i, 0)),
            pl.BlockSpec((block_rows, 128), lambda i: (i, 0)),
        ],
        out_shape=[
            jax.ShapeDtypeStruct((n, m), jnp.float32),
            jax.ShapeDtypeStruct((n, 128), jnp.float32),
        ],
    )(x, w, a2)


def _mlp_kernel(p_ref, wg_ref, bg_ref, w1_ref, b1_ref, w2_ref, b2_ref,
                wo_ref, bo_ref, o_ref):
    h = jnp.maximum(jnp.dot(p_ref[...], wg_ref[...],
                            preferred_element_type=jnp.float32)
                    + bg_ref[...], 0.0)
    h = jnp.maximum(jnp.dot(h, w1_ref[...],
                            preferred_element_type=jnp.float32)
                    + b1_ref[...], 0.0)
    h = jnp.maximum(jnp.dot(h, w2_ref[...],
                            preferred_element_type=jnp.float32)
                    + b2_ref[...], 0.0)
    o_ref[...] = jnp.dot(h, wo_ref[...],
                         preferred_element_type=jnp.float32) + bo_ref[...]


def _mlp(pooled, Wg, bg, Wf1, bf1, Wf2, bf2, Wo, bo):
    G = pooled.shape[0]
    Wo_pad = jnp.zeros((16, 128), jnp.float32).at[:, 0].set(Wo[:, 0])
    bo_pad = jnp.zeros((1, 128), jnp.float32).at[0, 0].set(bo[0])
    out = pl.pallas_call(
        _mlp_kernel,
        out_shape=jax.ShapeDtypeStruct((G, 128), jnp.float32),
    )(pooled, Wg, bg.reshape(1, -1), Wf1, bf1.reshape(1, -1),
      Wf2, bf2.reshape(1, -1), Wo_pad, bo_pad)
    return out[:, :1]


NPF = NP * 4


def _p1_body(svs, svd, srcp, dstp, cvec, *rest, heads):
    ee_outs = rest[:heads]
    den_out = rest[heads]
    scr = rest[heads + 1:]
    svs_v, svd_v, den_v, idx_s, idx_d, cv_b = scr[:6]
    ee_bufs = scr[6:6 + heads]
    c = lax.axis_index("c")
    s = lax.axis_index("s")
    tid = c * NS + s

    pltpu.sync_copy(svs, svs_v)
    pltpu.sync_copy(svd, svd_v)
    pltpu.sync_copy(cvec, cv_b)
    cv = cv_b[...]

    def zrow(i, _):
        den_v[pl.ds(16 * i, 16)] = jnp.zeros((16,), _f32)
        return 0
    lax.fori_loop(0, NPF // 16, zrow, 0)

    def step(st, _):
        gbase = (tid * U1 + st) * K
        pltpu.sync_copy(srcp.at[pl.ds(gbase, K)], idx_s)
        pltpu.sync_copy(dstp.at[pl.ds(gbase, K)], idx_d)
        for g in range(K // 16):
            sl = pl.ds(16 * g, 16)
            i_s4 = idx_s[sl] * 4
            i_d4 = idx_d[sl] * 4
            for h in range(heads):
                ss = plsc.load_gather(svs_v, [i_s4 + h])
                sd = plsc.load_gather(svd_v, [i_d4 + h])
                t = ss + sd
                e = jnp.where(t >= 0.0, t, 0.2 * t)
                ee = jnp.exp(e - cv)
                ee_bufs[h][sl] = ee
                plsc.addupdate_scatter(den_v, [i_d4 + h], ee)
        for h in range(heads):
            pltpu.sync_copy(ee_bufs[h], ee_outs[h].at[pl.ds(gbase, K)])
        return 0
    lax.fori_loop(0, U1, step, 0)
    pltpu.sync_copy(den_v, den_out.at[tid])


def _make_p1(heads):
    return pl.kernel(
        functools.partial(_p1_body, heads=heads),
        out_type=[jax.ShapeDtypeStruct((EP,), _f32)] * heads
        + [jax.ShapeDtypeStruct((NC * NS, NPF), _f32)],
        mesh=_MESH,
        scratch_types=[
            pltpu.VMEM((NPF,), _f32), pltpu.VMEM((NPF,), _f32),
            pltpu.VMEM((NPF,), _f32),
            pltpu.VMEM((K,), _i32), pltpu.VMEM((K,), _i32),
            pltpu.VMEM((16,), _f32),
        ] + [pltpu.VMEM((K,), _f32)] * heads,
        compiler_params=pltpu.CompilerParams(needs_layout_passes=False),
    )


_p1_h3 = _make_p1(3)
_p1_h1 = _make_p1(1)


def _p4_body(*refs, n_j, steps, split_cores):
    h_all = refs[0]
    pks = refs[1:1 + n_j]
    agg_out = refs[1 + n_j]
    scr = refs[2 + n_j:]
    pkb = scr[0:2]
    idx_a = scr[2:4]
    idx_d3 = scr[4]
    rows = scr[6:8]
    gsem = scr[8:10]
    ssem = scr[10:12]
    acc = scr[12]
    c = lax.axis_index("c")
    s = lax.axis_index("s")

    def gb(st):
        if split_cores:
            return ((c * NS + s) * steps + st) * K
        return (s * steps + st) * K

    for j in range(n_j):
        chunk = 2 * j + c if n_j > 1 else c * 0
        pk = pks[j]

        def zrow(i, _):
            for t in range(8):
                rows[0][i, pl.ds(16 * t, 16)] = jnp.zeros((16,), _f32)
            return 0
        lax.fori_loop(0, K, zrow, 0)
        for r in range(4):
            pltpu.sync_copy(rows[0], acc.at[pl.ds(s * ZR + r * K, K)])
        pltpu.sync_copy(rows[0].at[pl.ds(0, ZR - 4 * K)],
                        acc.at[pl.ds(s * ZR + 4 * K, ZR - 4 * K)])
        plsc.subcore_barrier()

        def prep(b, st, qn):
            pltpu.sync_copy(pk.at[:, pl.ds(gb(st), K)], pkb[b])
            for g16 in range(8):
                sl = pl.ds(16 * g16, 16)
                idx_a[b][sl] = pkb[b][0, sl] + chunk
                idx_d3[qn, sl] = pkb[b][1, sl]

        def gather(b):
            pltpu.async_copy(h_all.at[idx_a[b]], rows[b], gsem[b])

        def wait_g(b):
            pltpu.make_async_copy(h_all.at[idx_a[b]], rows[b],
                                  gsem[b]).wait()

        def scatter(b, q):
            pltpu.async_copy(rows[b], acc.at[idx_d3.at[q]], ssem[b],
                             add=True)

        def wait_s(b):
            pltpu.make_async_copy(rows[b], acc.at[idx_d3.at[0]],
                                  ssem[b]).wait()

        def compute(b):
            def wg(g16, _):
                wv = plsc.bitcast(pkb[b][2, pl.ds(16 * g16, 16)], _f32)
                for lane in range(16):
                    w = wv[lane]
                    k = g16 * 16 + lane
                    for t in range(8):
                        sl = pl.ds(16 * t, 16)
                        rows[b][k, sl] = rows[b][k, sl] * w
                return 0
            lax.fori_loop(0, 8, wg, 0)

        prep(0, 0, 0)
        gather(0)
        prep(1, 1, 1)

        def body2(i, _):
            q0 = lax.rem(2 * i, 4)
            wait_g(0)

            @pl.when(i > 0)
            def _():
                wait_s(1)
            gather(1)
            compute(0)
            scatter(0, q0)

            @pl.when(i < steps // 2 - 1)
            def _():
                prep(0, 2 * i + 2, lax.rem(2 * i + 2, 4))
            wait_g(1)
            wait_s(0)

            @pl.when(i < steps // 2 - 1)
            def _():
                gather(0)
            compute(1)
            scatter(1, lax.rem(2 * i + 1, 4))

            @pl.when(i < steps // 2 - 1)
            def _():
                prep(1, 2 * i + 3, lax.rem(2 * i + 3, 4))
            return 0
        lax.fori_loop(0, steps // 2, body2, 0)
        wait_s(1)
        plsc.subcore_barrier()
        if n_j > 1:
            pltpu.sync_copy(acc.at[pl.ds(s * ZR, ZR)],
                            agg_out.at[chunk, pl.ds(s * ZR, ZR)])
        else:
            pltpu.sync_copy(acc.at[pl.ds(s * ZR, ZR)],
                            agg_out.at[c, pl.ds(s * ZR, ZR)])


def _make_p4(n_j, steps, split_cores, out_shape):
    return pl.kernel(
        functools.partial(_p4_body, n_j=n_j, steps=steps,
                          split_cores=split_cores),
        out_type=jax.ShapeDtypeStruct(out_shape, _f32),
        mesh=_MESH,
        scratch_types=[
            pltpu.VMEM((3, K), _i32), pltpu.VMEM((3, K), _i32),
            pltpu.VMEM((K,), _i32), pltpu.VMEM((K,), _i32),
            pltpu.VMEM((4, K), _i32), pltpu.VMEM((K,), _i32),
            pltpu.VMEM((K, 128), _f32), pltpu.VMEM((K, 128), _f32),
            pltpu.SemaphoreType.DMA, pltpu.SemaphoreType.DMA,
            pltpu.SemaphoreType.DMA, pltpu.SemaphoreType.DMA,
            pltpu.VMEM_SHARED((NP, 128), _f32),
        ],
        compiler_params=pltpu.CompilerParams(needs_layout_passes=False),
    )


_p4_l1 = _make_p4(3, U4, False, (6, NP, 128))
_p4_l2 = _make_p4(1, U1, True, (NC, NP, 128))


def _edge_layer(h_all, s_scores, heads, srcp, dstp, p1, p4):
    s_src = s_scores[:, :heads]
    s_dst = s_scores[:, heads:2 * heads]
    C = jnp.max(s_src) + jnp.max(s_dst)
    cvec = jnp.full((16,), C, _f32)
    svs = jnp.pad(s_src, ((0, NP - N), (0, 4 - heads))).reshape(NPF)
    svd = jnp.pad(s_dst, ((0, NP - N), (0, 4 - heads))).reshape(NPF)
    outs = p1(svs, svd, srcp, dstp, cvec)
    ees, den_part = outs[:heads], outs[heads]
    den = den_part.sum(axis=0).reshape(NP, 4)[:N, :heads]
    mult = 6 if heads > 1 else 1
    src_m = srcp * mult
    pks = [jnp.stack([src_m, dstp, lax.bitcast_convert_type(ee, _i32)])
           for ee in ees]
    agg = p4(h_all, *pks)
    return agg, den


def kernel(x, edge_index, batch, W1, a_src1, a_dst1, b1, W2, a_src2,
           a_dst2, b2, Wg, bg, Wf1, bf1, Wf2, bf2, Wo, bo):
    G = 256
    H1 = a_src1.shape[0]
    loop = jnp.arange(N, dtype=edge_index.dtype)
    padi = jnp.full((EP - ETOT,), N, edge_index.dtype)
    srcp = jnp.concatenate([edge_index[0], loop, padi])
    dstp = jnp.concatenate([edge_index[1], loop, padi])

    a2_1 = jnp.stack([a_src1.reshape(-1), a_dst1.reshape(-1)])
    h1, s1 = _matmul_att(x, W1, a2_1, 1000, H1, x.shape[1])
    h_all1 = jnp.pad(h1, ((0, NP - N), (0, 0))).reshape(NP * 6, 128)
    agg1, den1s = _edge_layer(h_all1, s1, H1, srcp, dstp, _p1_h3, _p4_l1)
    out1 = agg1[:, :N].transpose(1, 0, 2).reshape(N, H1, 256)
    x1 = jax.nn.elu(out1 / (den1s[..., None] + 1e-16)
                    + b1.reshape(H1, 256)).reshape(N, H1 * 256)

    a2_2 = jnp.stack([a_src2.reshape(-1), a_dst2.reshape(-1)])
    h2, s2 = _matmul_att(x1, W2, a2_2, 1000, 1, W2.shape[1])
    h2p = jnp.pad(h2, ((0, NP - N), (0, 0)))
    agg2, den2s = _edge_layer(h2p, s2, 1, srcp, dstp, _p1_h1, _p4_l2)
    x2 = jax.nn.relu((agg2[0] + agg2[1])[:N] / (den2s + 1e-16) + b2)

    pooled = jax.ops.segment_max(x2, batch, num_segments=G)
    counts = jax.ops.segment_sum(jnp.ones((N,), jnp.float32), batch,
                                 num_segments=G)
    pooled = jnp.where(counts[:, None] > 0, pooled, 0.0)
    return _mlp(pooled, Wg, bg, Wf1, bf1, Wf2, bf2, Wo, bo)


# speedup vs baseline: 21.3712x; 1.1121x over previous
import functools

import jax
import jax.numpy as jnp
from jax import lax
from jax.experimental import pallas as pl
from jax.experimental.pallas import tpu as pltpu
from jax.experimental.pallas import tpu_sc as plsc

N = 10000
NP = 10112
ZR = NP // 16
E = 160000
ETOT = E + N
K = 128
EP = 172032
NC, NS = 2, 16
U1 = EP // (NC * NS * K)
U4 = EP // (NS * K)

_f32 = jnp.float32
_i32 = jnp.int32

_MESH = plsc.VectorSubcoreMesh(core_axis_name="c", subcore_axis_name="s")


def _mm_att_kernel(x_ref, w_ref, a_ref, h_ref, s_ref, *, heads, out_ch):
    h = jnp.dot(x_ref[...], w_ref[...], preferred_element_type=jnp.float32)
    h_ref[...] = h
    cols = []
    for k in range(2):
        for hd in range(heads):
            sl = slice(hd * out_ch, (hd + 1) * out_ch)
            prod = h[:, sl] * a_ref[k, sl][None, :]
            cols.append(jnp.sum(prod, axis=1, keepdims=True))
    s = jnp.concatenate(cols, axis=1)
    s_ref[...] = jnp.pad(s, ((0, 0), (0, 128 - 2 * heads)))


def _matmul_att(x, w, a2, block_rows, heads, out_ch):
    n, k = x.shape
    _, m = w.shape
    grid = n // block_rows
    return pl.pallas_call(
        functools.partial(_mm_att_kernel, heads=heads, out_ch=out_ch),
        grid=(grid,),
        in_specs=[
            pl.BlockSpec((block_rows, k), lambda i: (i, 0)),
            pl.BlockSpec((k, m), lambda i: (0, 0)),
            pl.BlockSpec((2, m), lambda i: (0, 0)),
        ],
        out_specs=[
            pl.BlockSpec((block_rows, m), lambda i: (i, 0)),
            pl.BlockSpec((block_rows, 128), lambda i: (i, 0)),
        ],
        out_shape=[
            jax.ShapeDtypeStruct((n, m), jnp.float32),
            jax.ShapeDtypeStruct((n, 128), jnp.float32),
        ],
    )(x, w, a2)


def _l2_kernel(a_ref, den_ref, b1_ref, w_ref, a2_ref, h_ref, s_ref):
    i = pl.program_id(0)
    rid = i * 632 + lax.broadcasted_iota(jnp.int32, (632, 1), 0)
    valid = rid < N
    h2 = jnp.zeros((632, 128), jnp.float32)
    for c in range(6):
        dh = den_ref[:, c // 2:c // 2 + 1]
        xc = a_ref[c] / (dh + 1e-16) + b1_ref[0, 128 * c:128 * (c + 1)]
        xc = jnp.where(xc > 0, xc, jnp.exp(jnp.minimum(xc, 0.0)) - 1.0)
        xc = jnp.where(valid, xc, 0.0)
        h2 = h2 + jnp.dot(xc, w_ref[128 * c:128 * (c + 1), :],
                          preferred_element_type=jnp.float32)
    h_ref[...] = h2
    ss = jnp.sum(h2 * a2_ref[0][None, :], axis=1, keepdims=True)
    sd = jnp.sum(h2 * a2_ref[1][None, :], axis=1, keepdims=True)
    s_ref[...] = jnp.pad(jnp.concatenate([ss, sd], axis=1),
                         ((0, 0), (0, 126)))


def _l2_fused(agg1, den128, b1, W2, a2_2):
    grid = NP // 632
    return pl.pallas_call(
        _l2_kernel,
        grid=(grid,),
        in_specs=[
            pl.BlockSpec((6, 632, 128), lambda i: (0, i, 0)),
            pl.BlockSpec((632, 128), lambda i: (i, 0)),
            pl.BlockSpec((1, 768), lambda i: (0, 0)),
            pl.BlockSpec((768, 128), lambda i: (0, 0)),
            pl.BlockSpec((2, 128), lambda i: (0, 0)),
        ],
        out_specs=[
            pl.BlockSpec((632, 128), lambda i: (i, 0)),
            pl.BlockSpec((632, 128), lambda i: (i, 0)),
        ],
        out_shape=[
            jax.ShapeDtypeStruct((NP, 128), jnp.float32),
            jax.ShapeDtypeStruct((NP, 128), jnp.float32),
        ],
    )(agg1, den128, b1.reshape(1, -1), W2, a2_2)


def _mlp_kernel(p_ref, wg_ref, bg_ref, w1_ref, b1_ref, w2_ref, b2_ref,
                wo_ref, bo_ref, o_ref):
    h = jnp.maximum(jnp.dot(p_ref[...], wg_ref[...],
                            preferred_element_type=jnp.float32)
                    + bg_ref[...], 0.0)
    h = jnp.maximum(jnp.dot(h, w1_ref[...],
                            preferred_element_type=jnp.float32)
                    + b1_ref[...], 0.0)
    h = jnp.maximum(jnp.dot(h, w2_ref[...],
                            preferred_element_type=jnp.float32)
                    + b2_ref[...], 0.0)
    o_ref[...] = jnp.dot(h, wo_ref[...],
                         preferred_element_type=jnp.float32) + bo_ref[...]


def _mlp(pooled, Wg, bg, Wf1, bf1, Wf2, bf2, Wo, bo):
    G = pooled.shape[0]
    Wo_pad = jnp.zeros((16, 128), jnp.float32).at[:, 0].set(Wo[:, 0])
    bo_pad = jnp.zeros((1, 128), jnp.float32).at[0, 0].set(bo[0])
    out = pl.pallas_call(
        _mlp_kernel,
        out_shape=jax.ShapeDtypeStruct((G, 128), jnp.float32),
    )(pooled, Wg, bg.reshape(1, -1), Wf1, bf1.reshape(1, -1),
      Wf2, bf2.reshape(1, -1), Wo_pad, bo_pad)
    return out[:, :1]


NPF = NP * 4


def _p1_body(svs, svd, srcp, dstp, cvec, *rest, heads):
    ee_outs = rest[:heads]
    den_out = rest[heads]
    scr = rest[heads + 1:]
    svs_v, svd_v, den_v, idx_s, idx_d, cv_b = scr[:6]
    ee_bufs = scr[6:6 + heads]
    c = lax.axis_index("c")
    s = lax.axis_index("s")
    tid = c * NS + s

    pltpu.sync_copy(svs, svs_v)
    pltpu.sync_copy(svd, svd_v)
    pltpu.sync_copy(cvec, cv_b)
    cv = cv_b[...]

    def zrow(i, _):
        den_v[pl.ds(16 * i, 16)] = jnp.zeros((16,), _f32)
        return 0
    lax.fori_loop(0, NPF // 16, zrow, 0)

    def step(st, _):
        gbase = (tid * U1 + st) * K
        pltpu.sync_copy(srcp.at[pl.ds(gbase, K)], idx_s)
        pltpu.sync_copy(dstp.at[pl.ds(gbase, K)], idx_d)
        for g in range(K // 16):
            sl = pl.ds(16 * g, 16)
            i_s4 = idx_s[sl] * 4
            i_d4 = idx_d[sl] * 4
            for h in range(heads):
                ss = plsc.load_gather(svs_v, [i_s4 + h])
                sd = plsc.load_gather(svd_v, [i_d4 + h])
                t = ss + sd
                e = jnp.where(t >= 0.0, t, 0.2 * t)
                ee = jnp.exp(e - cv)
                ee_bufs[h][sl] = ee
                plsc.addupdate_scatter(den_v, [i_d4 + h], ee)
        for h in range(heads):
            pltpu.sync_copy(ee_bufs[h], ee_outs[h].at[pl.ds(gbase, K)])
        return 0
    lax.fori_loop(0, U1, step, 0)
    pltpu.sync_copy(den_v, den_out.at[tid])


def _make_p1(heads):
    return pl.kernel(
        functools.partial(_p1_body, heads=heads),
        out_type=[jax.ShapeDtypeStruct((EP,), _f32)] * heads
        + [jax.ShapeDtypeStruct((NC * NS, NPF), _f32)],
        mesh=_MESH,
        scratch_types=[
            pltpu.VMEM((NPF,), _f32), pltpu.VMEM((NPF,), _f32),
            pltpu.VMEM((NPF,), _f32),
            pltpu.VMEM((K,), _i32), pltpu.VMEM((K,), _i32),
            pltpu.VMEM((16,), _f32),
        ] + [pltpu.VMEM((K,), _f32)] * heads,
        compiler_params=pltpu.CompilerParams(needs_layout_passes=False),
    )


_p1_h3 = _make_p1(3)
_p1_h1 = _make_p1(1)


def _p4_body(*refs, n_j, steps, split_cores):
    h_all = refs[0]
    pks = refs[1:1 + n_j]
    agg_out = refs[1 + n_j]
    scr = refs[2 + n_j:]
    pkb = scr[0:2]
    idx_a = scr[2:4]
    idx_d3 = scr[4]
    rows = scr[6:8]
    gsem = scr[8:10]
    ssem = scr[10:12]
    acc = scr[12]
    c = lax.axis_index("c")
    s = lax.axis_index("s")

    def gb(st):
        if split_cores:
            return ((c * NS + s) * steps + st) * K
        return (s * steps + st) * K

    for j in range(n_j):
        chunk = 2 * j + c if n_j > 1 else c * 0
        pk = pks[j]

        def zrow(i, _):
            for t in range(8):
                rows[0][i, pl.ds(16 * t, 16)] = jnp.zeros((16,), _f32)
            return 0
        lax.fori_loop(0, K, zrow, 0)
        for r in range(4):
            pltpu.sync_copy(rows[0], acc.at[pl.ds(s * ZR + r * K, K)])
        pltpu.sync_copy(rows[0].at[pl.ds(0, ZR - 4 * K)],
                        acc.at[pl.ds(s * ZR + 4 * K, ZR - 4 * K)])
        plsc.subcore_barrier()

        def prep(b, st, qn):
            pltpu.sync_copy(pk.at[:, pl.ds(gb(st), K)], pkb[b])
            for g16 in range(8):
                sl = pl.ds(16 * g16, 16)
                idx_a[b][sl] = pkb[b][0, sl] + chunk
                idx_d3[qn, sl] = pkb[b][1, sl]

        def gather(b):
            pltpu.async_copy(h_all.at[idx_a[b]], rows[b], gsem[b])

        def wait_g(b):
            pltpu.make_async_copy(h_all.at[idx_a[b]], rows[b],
                                  gsem[b]).wait()

        def scatter(b, q):
            pltpu.async_copy(rows[b], acc.at[idx_d3.at[q]], ssem[b],
                             add=True)

        def wait_s(b):
            pltpu.make_async_copy(rows[b], acc.at[idx_d3.at[0]],
                                  ssem[b]).wait()

        def compute(b):
            def wg(g16, _):
                wv = plsc.bitcast(pkb[b][2, pl.ds(16 * g16, 16)], _f32)
                for lane in range(16):
                    w = wv[lane]
                    k = g16 * 16 + lane
                    for t in range(8):
                        sl = pl.ds(16 * t, 16)
                        rows[b][k, sl] = rows[b][k, sl] * w
                return 0
            lax.fori_loop(0, 8, wg, 0)

        prep(0, 0, 0)
        gather(0)
        prep(1, 1, 1)

        def body2(i, _):
            q0 = lax.rem(2 * i, 4)
            wait_g(0)

            @pl.when(i > 0)
            def _():
                wait_s(1)
            gather(1)
            compute(0)
            scatter(0, q0)

            @pl.when(i < steps // 2 - 1)
            def _():
                prep(0, 2 * i + 2, lax.rem(2 * i + 2, 4))
            wait_g(1)
            wait_s(0)

            @pl.when(i < steps // 2 - 1)
            def _():
                gather(0)
            compute(1)
            scatter(1, lax.rem(2 * i + 1, 4))

            @pl.when(i < steps // 2 - 1)
            def _():
                prep(1, 2 * i + 3, lax.rem(2 * i + 3, 4))
            return 0
        lax.fori_loop(0, steps // 2, body2, 0)
        wait_s(1)
        plsc.subcore_barrier()
        if n_j > 1:
            pltpu.sync_copy(acc.at[pl.ds(s * ZR, ZR)],
                            agg_out.at[chunk, pl.ds(s * ZR, ZR)])
        else:
            pltpu.sync_copy(acc.at[pl.ds(s * ZR, ZR)],
                            agg_out.at[c, pl.ds(s * ZR, ZR)])


def _make_p4(n_j, steps, split_cores, out_shape):
    return pl.kernel(
        functools.partial(_p4_body, n_j=n_j, steps=steps,
                          split_cores=split_cores),
        out_type=jax.ShapeDtypeStruct(out_shape, _f32),
        mesh=_MESH,
        scratch_types=[
            pltpu.VMEM((3, K), _i32), pltpu.VMEM((3, K), _i32),
            pltpu.VMEM((K,), _i32), pltpu.VMEM((K,), _i32),
            pltpu.VMEM((4, K), _i32), pltpu.VMEM((K,), _i32),
            pltpu.VMEM((K, 128), _f32), pltpu.VMEM((K, 128), _f32),
            pltpu.SemaphoreType.DMA, pltpu.SemaphoreType.DMA,
            pltpu.SemaphoreType.DMA, pltpu.SemaphoreType.DMA,
            pltpu.VMEM_SHARED((NP, 128), _f32),
        ],
        compiler_params=pltpu.CompilerParams(needs_layout_passes=False),
    )


_p4_l1 = _make_p4(3, U4, False, (6, NP, 128))
_p4_l2 = _make_p4(1, U1, True, (NC, NP, 128))


def _edge_layer(h_all, s_scores, heads, srcp, dstp, p1, p4):
    s_src = s_scores[:, :heads]
    s_dst = s_scores[:, heads:2 * heads]
    C = jnp.max(s_src) + jnp.max(s_dst)
    cvec = jnp.full((16,), C, _f32)
    svs = jnp.pad(s_src, ((0, 0), (0, 4 - heads))).reshape(NPF)
    svd = jnp.pad(s_dst, ((0, 0), (0, 4 - heads))).reshape(NPF)
    outs = p1(svs, svd, srcp, dstp, cvec)
    ees, den_part = outs[:heads], outs[heads]
    den = den_part.sum(axis=0).reshape(NP, 4)
    mult = 6 if heads > 1 else 1
    src_m = srcp * mult
    pks = [jnp.stack([src_m, dstp, lax.bitcast_convert_type(ee, _i32)])
           for ee in ees]
    agg = p4(h_all, *pks)
    return agg, den


def kernel(x, edge_index, batch, W1, a_src1, a_dst1, b1, W2, a_src2,
           a_dst2, b2, Wg, bg, Wf1, bf1, Wf2, bf2, Wo, bo):
    G = 256
    H1 = a_src1.shape[0]
    loop = jnp.arange(N, dtype=edge_index.dtype)
    padi = jnp.full((EP - ETOT,), N, edge_index.dtype)
    srcp = jnp.concatenate([edge_index[0], loop, padi])
    dstp = jnp.concatenate([edge_index[1], loop, padi])

    a2_1 = jnp.stack([a_src1.reshape(-1), a_dst1.reshape(-1)])
    xp = jnp.pad(x, ((0, NP - N), (0, 0)))
    h1, s1 = _matmul_att(xp, W1, a2_1, 632, H1, x.shape[1])
    h_all1 = h1.reshape(NP * 6, 128)
    agg1, den4_1 = _edge_layer(h_all1, s1, H1, srcp, dstp, _p1_h3, _p4_l1)

    a2_2 = jnp.stack([a_src2.reshape(-1), a_dst2.reshape(-1)])
    den128 = jnp.pad(den4_1, ((0, 0), (0, 124)))
    h2, s2 = _l2_fused(agg1, den128, b1, W2, a2_2)
    agg2, den4_2 = _edge_layer(h2, s2, 1, srcp, dstp, _p1_h1, _p4_l2)
    den2s = den4_2[:N, :1]
    x2 = jax.nn.relu((agg2[0] + agg2[1])[:N] / (den2s + 1e-16) + b2)

    pooled = jax.ops.segment_max(x2, batch, num_segments=G)
    counts = jax.ops.segment_sum(jnp.ones((N,), jnp.float32), batch,
                                 num_segments=G)
    pooled = jnp.where(counts[:, None] > 0, pooled, 0.0)
    return _mlp(pooled, Wg, bg, Wf1, bf1, Wf2, bf2, Wo, bo)


# speedup vs baseline: 22.1407x; 1.0360x over previous
import functools

import jax
import jax.numpy as jnp
from jax import lax
from jax.experimental import pallas as pl
from jax.experimental.pallas import tpu as pltpu
from jax.experimental.pallas import tpu_sc as plsc

N = 10000
NP = 10112
ZR = NP // 16
E = 160000
ETOT = E + N
K = 128
EP = 172032
NC, NS = 2, 16
U1 = EP // (NC * NS * K)
U4 = EP // (NS * K)

_f32 = jnp.float32
_i32 = jnp.int32

_MESH = plsc.VectorSubcoreMesh(core_axis_name="c", subcore_axis_name="s")


def _mm_att_kernel(x_ref, w_ref, a_ref, h_ref, s_ref, *, heads, out_ch):
    h = jnp.dot(x_ref[...], w_ref[...], preferred_element_type=jnp.float32)
    h_ref[...] = h
    cols = []
    for k in range(2):
        for hd in range(heads):
            sl = slice(hd * out_ch, (hd + 1) * out_ch)
            prod = h[:, sl] * a_ref[k, sl][None, :]
            cols.append(jnp.sum(prod, axis=1, keepdims=True))
    s = jnp.concatenate(cols, axis=1)
    s_ref[...] = jnp.pad(s, ((0, 0), (0, 128 - 2 * heads)))


def _matmul_att(x, w, a2, block_rows, heads, out_ch):
    n, k = x.shape
    _, m = w.shape
    grid = n // block_rows
    return pl.pallas_call(
        functools.partial(_mm_att_kernel, heads=heads, out_ch=out_ch),
        grid=(grid,),
        in_specs=[
            pl.BlockSpec((block_rows, k), lambda i: (i, 0)),
            pl.BlockSpec((k, m), lambda i: (0, 0)),
            pl.BlockSpec((2, m), lambda i: (0, 0)),
        ],
        out_specs=[
            pl.BlockSpec((block_rows, m), lambda i: (i, 0)),
            pl.BlockSpec((block_rows, 128), lambda i: (i, 0)),
        ],
        out_shape=[
            jax.ShapeDtypeStruct((n, m), jnp.float32),
            jax.ShapeDtypeStruct((n, 128), jnp.float32),
        ],
    )(x, w, a2)


def _l2_kernel(a_ref, den_ref, b1_ref, w_ref, a2_ref, h_ref, s_ref):
    i = pl.program_id(0)
    rid = i * 632 + lax.broadcasted_iota(jnp.int32, (632, 1), 0)
    valid = rid < N
    h2 = jnp.zeros((632, 128), jnp.float32)
    for c in range(6):
        dh = den_ref[:, c // 2:c // 2 + 1]
        xc = a_ref[c] / (dh + 1e-16) + b1_ref[0, 128 * c:128 * (c + 1)]
        xc = jnp.where(xc > 0, xc, jnp.exp(jnp.minimum(xc, 0.0)) - 1.0)
        xc = jnp.where(valid, xc, 0.0)
        h2 = h2 + jnp.dot(xc, w_ref[128 * c:128 * (c + 1), :],
                          preferred_element_type=jnp.float32)
    h_ref[...] = h2
    ss = jnp.sum(h2 * a2_ref[0][None, :], axis=1, keepdims=True)
    sd = jnp.sum(h2 * a2_ref[1][None, :], axis=1, keepdims=True)
    s_ref[...] = jnp.pad(jnp.concatenate([ss, sd], axis=1),
                         ((0, 0), (0, 126)))


def _l2_fused(agg1, den128, b1, W2, a2_2):
    grid = NP // 632
    return pl.pallas_call(
        _l2_kernel,
        grid=(grid,),
        in_specs=[
            pl.BlockSpec((6, 632, 128), lambda i: (0, i, 0)),
            pl.BlockSpec((632, 128), lambda i: (i, 0)),
            pl.BlockSpec((1, 768), lambda i: (0, 0)),
            pl.BlockSpec((768, 128), lambda i: (0, 0)),
            pl.BlockSpec((2, 128), lambda i: (0, 0)),
        ],
        out_specs=[
            pl.BlockSpec((632, 128), lambda i: (i, 0)),
            pl.BlockSpec((632, 128), lambda i: (i, 0)),
        ],
        out_shape=[
            jax.ShapeDtypeStruct((NP, 128), jnp.float32),
            jax.ShapeDtypeStruct((NP, 128), jnp.float32),
        ],
    )(agg1, den128, b1.reshape(1, -1), W2, a2_2)


def _mlp_kernel(p_ref, wg_ref, bg_ref, w1_ref, b1_ref, w2_ref, b2_ref,
                wo_ref, bo_ref, o_ref):
    h = jnp.maximum(jnp.dot(p_ref[...], wg_ref[...],
                            preferred_element_type=jnp.float32)
                    + bg_ref[...], 0.0)
    h = jnp.maximum(jnp.dot(h, w1_ref[...],
                            preferred_element_type=jnp.float32)
                    + b1_ref[...], 0.0)
    h = jnp.maximum(jnp.dot(h, w2_ref[...],
                            preferred_element_type=jnp.float32)
                    + b2_ref[...], 0.0)
    o_ref[...] = jnp.dot(h, wo_ref[...],
                         preferred_element_type=jnp.float32) + bo_ref[...]


def _mlp(pooled, Wg, bg, Wf1, bf1, Wf2, bf2, Wo, bo):
    G = pooled.shape[0]
    Wo_pad = jnp.zeros((16, 128), jnp.float32).at[:, 0].set(Wo[:, 0])
    bo_pad = jnp.zeros((1, 128), jnp.float32).at[0, 0].set(bo[0])
    out = pl.pallas_call(
        _mlp_kernel,
        out_shape=jax.ShapeDtypeStruct((G, 128), jnp.float32),
    )(pooled, Wg, bg.reshape(1, -1), Wf1, bf1.reshape(1, -1),
      Wf2, bf2.reshape(1, -1), Wo_pad, bo_pad)
    return out[:, :1]


NPF = NP * 4


def _p1_body(svs, svd, sd2, cvec, ee_all, den_out,
             svs_v, svd_v, den_v, pk_b, ee_b, cv_b, *, heads):
    c = lax.axis_index("c")
    s = lax.axis_index("s")
    tid = c * NS + s

    pltpu.sync_copy(svs, svs_v)
    pltpu.sync_copy(svd, svd_v)
    pltpu.sync_copy(cvec, cv_b)
    cv = cv_b[...]

    def zrow(i, _):
        den_v[pl.ds(16 * i, 16)] = jnp.zeros((16,), _f32)
        return 0
    lax.fori_loop(0, NPF // 16, zrow, 0)

    def step(st, _):
        gbase = (tid * U1 + st) * K
        pltpu.sync_copy(sd2.at[:, pl.ds(gbase, K)], pk_b)
        for g in range(K // 16):
            sl = pl.ds(16 * g, 16)
            i_s4 = pk_b[0, sl] * 4
            i_d4 = pk_b[1, sl] * 4
            for h in range(heads):
                ss = plsc.load_gather(svs_v, [i_s4 + h])
                sd = plsc.load_gather(svd_v, [i_d4 + h])
                t = ss + sd
                e = jnp.where(t >= 0.0, t, 0.2 * t)
                ee = jnp.exp(e - cv)
                ee_b[h, sl] = ee
                plsc.addupdate_scatter(den_v, [i_d4 + h], ee)
        pltpu.sync_copy(ee_b, ee_all.at[:, pl.ds(gbase, K)])
        return 0
    lax.fori_loop(0, U1, step, 0)
    pltpu.sync_copy(den_v, den_out.at[tid])


def _make_p1(heads):
    return pl.kernel(
        functools.partial(_p1_body, heads=heads),
        out_type=[jax.ShapeDtypeStruct((4, EP), _f32),
                  jax.ShapeDtypeStruct((NC * NS, NPF), _f32)],
        mesh=_MESH,
        scratch_types=[
            pltpu.VMEM((NPF,), _f32), pltpu.VMEM((NPF,), _f32),
            pltpu.VMEM((NPF,), _f32),
            pltpu.VMEM((2, K), _i32), pltpu.VMEM((4, K), _f32),
            pltpu.VMEM((16,), _f32),
        ],
        compiler_params=pltpu.CompilerParams(needs_layout_passes=False),
    )


_p1_h3 = _make_p1(3)
_p1_h1 = _make_p1(1)


def _p4_body(*refs, n_j, steps, split_cores):
    h_all = refs[0]
    pks = refs[1:1 + n_j]
    agg_out = refs[1 + n_j]
    scr = refs[2 + n_j:]
    pkb = scr[0:2]
    idx_a = scr[2:4]
    idx_d3 = scr[4]
    rows = scr[6:8]
    gsem = scr[8:10]
    ssem = scr[10:12]
    acc = scr[12]
    c = lax.axis_index("c")
    s = lax.axis_index("s")

    def gb(st):
        if split_cores:
            return ((c * NS + s) * steps + st) * K
        return (s * steps + st) * K

    for j in range(n_j):
        chunk = 2 * j + c if n_j > 1 else c * 0
        pk = pks[j]

        def zrow(i, _):
            for t in range(8):
                rows[0][i, pl.ds(16 * t, 16)] = jnp.zeros((16,), _f32)
            return 0
        lax.fori_loop(0, K, zrow, 0)
        for r in range(4):
            pltpu.sync_copy(rows[0], acc.at[pl.ds(s * ZR + r * K, K)])
        pltpu.sync_copy(rows[0].at[pl.ds(0, ZR - 4 * K)],
                        acc.at[pl.ds(s * ZR + 4 * K, ZR - 4 * K)])
        plsc.subcore_barrier()

        def prep(b, st, qn):
            pltpu.sync_copy(pk.at[:, pl.ds(gb(st), K)], pkb[b])
            for g16 in range(8):
                sl = pl.ds(16 * g16, 16)
                idx_a[b][sl] = pkb[b][0, sl] + chunk
                idx_d3[qn, sl] = pkb[b][1, sl]

        def gather(b):
            pltpu.async_copy(h_all.at[idx_a[b]], rows[b], gsem[b])

        def wait_g(b):
            pltpu.make_async_copy(h_all.at[idx_a[b]], rows[b],
                                  gsem[b]).wait()

        def scatter(b, q):
            pltpu.async_copy(rows[b], acc.at[idx_d3.at[q]], ssem[b],
                             add=True)

        def wait_s(b):
            pltpu.make_async_copy(rows[b], acc.at[idx_d3.at[0]],
                                  ssem[b]).wait()

        def compute(b):
            def wg(g16, _):
                wv = plsc.bitcast(pkb[b][2, pl.ds(16 * g16, 16)], _f32)
                for lane in range(16):
                    w = wv[lane]
                    k = g16 * 16 + lane
                    for t in range(8):
                        sl = pl.ds(16 * t, 16)
                        rows[b][k, sl] = rows[b][k, sl] * w
                return 0
            lax.fori_loop(0, 8, wg, 0)

        prep(0, 0, 0)
        gather(0)
        prep(1, 1, 1)

        def body2(i, _):
            q0 = lax.rem(2 * i, 4)
            wait_g(0)

            @pl.when(i > 0)
            def _():
                wait_s(1)
            gather(1)
            compute(0)
            scatter(0, q0)

            @pl.when(i < steps // 2 - 1)
            def _():
                prep(0, 2 * i + 2, lax.rem(2 * i + 2, 4))
            wait_g(1)
            wait_s(0)

            @pl.when(i < steps // 2 - 1)
            def _():
                gather(0)
            compute(1)
            scatter(1, lax.rem(2 * i + 1, 4))

            @pl.when(i < steps // 2 - 1)
            def _():
                prep(1, 2 * i + 3, lax.rem(2 * i + 3, 4))
            return 0
        lax.fori_loop(0, steps // 2, body2, 0)
        wait_s(1)
        plsc.subcore_barrier()
        if n_j > 1:
            pltpu.sync_copy(acc.at[pl.ds(s * ZR, ZR)],
                            agg_out.at[chunk, pl.ds(s * ZR, ZR)])
        else:
            pltpu.sync_copy(acc.at[pl.ds(s * ZR, ZR)],
                            agg_out.at[c, pl.ds(s * ZR, ZR)])


def _make_p4(n_j, steps, split_cores, out_shape):
    return pl.kernel(
        functools.partial(_p4_body, n_j=n_j, steps=steps,
                          split_cores=split_cores),
        out_type=jax.ShapeDtypeStruct(out_shape, _f32),
        mesh=_MESH,
        scratch_types=[
            pltpu.VMEM((3, K), _i32), pltpu.VMEM((3, K), _i32),
            pltpu.VMEM((K,), _i32), pltpu.VMEM((K,), _i32),
            pltpu.VMEM((4, K), _i32), pltpu.VMEM((K,), _i32),
            pltpu.VMEM((K, 128), _f32), pltpu.VMEM((K, 128), _f32),
            pltpu.SemaphoreType.DMA, pltpu.SemaphoreType.DMA,
            pltpu.SemaphoreType.DMA, pltpu.SemaphoreType.DMA,
            pltpu.VMEM_SHARED((NP, 128), _f32),
        ],
        compiler_params=pltpu.CompilerParams(needs_layout_passes=False),
    )


_p4_l1 = _make_p4(3, U4, False, (6, NP, 128))
_p4_l2 = _make_p4(1, U1, True, (NC, NP, 128))


def _edge_layer(h_all, s_scores, heads, srcp, dstp, p1, p4):
    s_src = s_scores[:, :heads]
    s_dst = s_scores[:, heads:2 * heads]
    C = jnp.max(s_src) + jnp.max(s_dst)
    cvec = jnp.full((16,), C, _f32)
    svs = jnp.pad(s_src, ((0, 0), (0, 4 - heads))).reshape(NPF)
    svd = jnp.pad(s_dst, ((0, 0), (0, 4 - heads))).reshape(NPF)
    sd2 = jnp.stack([srcp, dstp])
    ee_all, den_part = p1(svs, svd, sd2, cvec)
    den = den_part.sum(axis=0).reshape(NP, 4)
    mult = 6 if heads > 1 else 1
    src_m = srcp * mult
    pks = [jnp.stack([src_m, dstp,
                      lax.bitcast_convert_type(ee_all[h], _i32)])
           for h in range(heads)]
    agg = p4(h_all, *pks)
    return agg, den


def kernel(x, edge_index, batch, W1, a_src1, a_dst1, b1, W2, a_src2,
           a_dst2, b2, Wg, bg, Wf1, bf1, Wf2, bf2, Wo, bo):
    G = 256
    H1 = a_src1.shape[0]
    loop = jnp.arange(N, dtype=edge_index.dtype)
    padi = jnp.full((EP - ETOT,), N, edge_index.dtype)
    srcp = jnp.concatenate([edge_index[0], loop, padi])
    dstp = jnp.concatenate([edge_index[1], loop, padi])

    a2_1 = jnp.stack([a_src1.reshape(-1), a_dst1.reshape(-1)])
    xp = jnp.pad(x, ((0, NP - N), (0, 0)))
    h1, s1 = _matmul_att(xp, W1, a2_1, 632, H1, x.shape[1])
    h_all1 = h1.reshape(NP * 6, 128)
    agg1, den4_1 = _edge_layer(h_all1, s1, H1, srcp, dstp, _p1_h3, _p4_l1)

    a2_2 = jnp.stack([a_src2.reshape(-1), a_dst2.reshape(-1)])
    den128 = jnp.pad(den4_1, ((0, 0), (0, 124)))
    h2, s2 = _l2_fused(agg1, den128, b1, W2, a2_2)
    agg2, den4_2 = _edge_layer(h2, s2, 1, srcp, dstp, _p1_h1, _p4_l2)
    den2s = den4_2[:N, :1]
    x2 = jax.nn.relu((agg2[0] + agg2[1])[:N] / (den2s + 1e-16) + b2)

    pooled = jax.ops.segment_max(x2, batch, num_segments=G)
    counts = jax.ops.segment_sum(jnp.ones((N,), jnp.float32), batch,
                                 num_segments=G)
    pooled = jnp.where(counts[:, None] > 0, pooled, 0.0)
    return _mlp(pooled, Wg, bg, Wf1, bf1, Wf2, bf2, Wo, bo)


# speedup vs baseline: 22.5904x; 1.0203x over previous
import functools

import jax
import jax.numpy as jnp
from jax import lax
from jax.experimental import pallas as pl
from jax.experimental.pallas import tpu as pltpu
from jax.experimental.pallas import tpu_sc as plsc

N = 10000
NP = 10112
ZR = NP // 16
E = 160000
ETOT = E + N
K = 128
EP = 172032
NC, NS = 2, 16
U1 = EP // (NC * NS * K)
U4 = EP // (NS * K)

_f32 = jnp.float32
_i32 = jnp.int32

_MESH = plsc.VectorSubcoreMesh(core_axis_name="c", subcore_axis_name="s")


def _mm_att_kernel(x_ref, w_ref, a_ref, h_ref, s_ref, *, heads, out_ch):
    h = jnp.dot(x_ref[...], w_ref[...], preferred_element_type=jnp.float32)
    h_ref[...] = h
    cols = []
    for k in range(2):
        for hd in range(heads):
            sl = slice(hd * out_ch, (hd + 1) * out_ch)
            prod = h[:, sl] * a_ref[k, sl][None, :]
            cols.append(jnp.sum(prod, axis=1, keepdims=True))
    s = jnp.concatenate(cols, axis=1)
    s_ref[...] = jnp.pad(s, ((0, 0), (0, 128 - 2 * heads)))


def _matmul_att(x, w, a2, block_rows, heads, out_ch):
    n, k = x.shape
    _, m = w.shape
    grid = n // block_rows
    return pl.pallas_call(
        functools.partial(_mm_att_kernel, heads=heads, out_ch=out_ch),
        grid=(grid,),
        in_specs=[
            pl.BlockSpec((block_rows, k), lambda i: (i, 0)),
            pl.BlockSpec((k, m), lambda i: (0, 0)),
            pl.BlockSpec((2, m), lambda i: (0, 0)),
        ],
        out_specs=[
            pl.BlockSpec((block_rows, m), lambda i: (i, 0)),
            pl.BlockSpec((block_rows, 128), lambda i: (i, 0)),
        ],
        out_shape=[
            jax.ShapeDtypeStruct((n, m), jnp.float32),
            jax.ShapeDtypeStruct((n, 128), jnp.float32),
        ],
    )(x, w, a2)


def _l2_kernel(a_ref, den_ref, b1_ref, w_ref, a2_ref, h_ref, s_ref):
    i = pl.program_id(0)
    rid = i * 632 + lax.broadcasted_iota(jnp.int32, (632, 1), 0)
    valid = rid < N
    h2 = jnp.zeros((632, 128), jnp.float32)
    for c in range(6):
        dh = den_ref[:, c // 2:c // 2 + 1]
        xc = a_ref[c] / (dh + 1e-16) + b1_ref[0, 128 * c:128 * (c + 1)]
        xc = jnp.where(xc > 0, xc, jnp.exp(jnp.minimum(xc, 0.0)) - 1.0)
        xc = jnp.where(valid, xc, 0.0)
        h2 = h2 + jnp.dot(xc, w_ref[128 * c:128 * (c + 1), :],
                          preferred_element_type=jnp.float32)
    h_ref[...] = h2
    ss = jnp.sum(h2 * a2_ref[0][None, :], axis=1, keepdims=True)
    sd = jnp.sum(h2 * a2_ref[1][None, :], axis=1, keepdims=True)
    s_ref[...] = jnp.pad(jnp.concatenate([ss, sd], axis=1),
                         ((0, 0), (0, 126)))


def _l2_fused(agg1, den128, b1, W2, a2_2):
    grid = NP // 632
    return pl.pallas_call(
        _l2_kernel,
        grid=(grid,),
        in_specs=[
            pl.BlockSpec((6, 632, 128), lambda i: (0, i, 0)),
            pl.BlockSpec((632, 128), lambda i: (i, 0)),
            pl.BlockSpec((1, 768), lambda i: (0, 0)),
            pl.BlockSpec((768, 128), lambda i: (0, 0)),
            pl.BlockSpec((2, 128), lambda i: (0, 0)),
        ],
        out_specs=[
            pl.BlockSpec((632, 128), lambda i: (i, 0)),
            pl.BlockSpec((632, 128), lambda i: (i, 0)),
        ],
        out_shape=[
            jax.ShapeDtypeStruct((NP, 128), jnp.float32),
            jax.ShapeDtypeStruct((NP, 128), jnp.float32),
        ],
    )(agg1, den128, b1.reshape(1, -1), W2, a2_2)


def _mlp_kernel(p_ref, wg_ref, bg_ref, w1_ref, b1_ref, w2_ref, b2_ref,
                wo_ref, bo_ref, o_ref):
    h = jnp.maximum(jnp.dot(p_ref[...], wg_ref[...],
                            preferred_element_type=jnp.float32)
                    + bg_ref[...], 0.0)
    h = jnp.maximum(jnp.dot(h, w1_ref[...],
                            preferred_element_type=jnp.float32)
                    + b1_ref[...], 0.0)
    h = jnp.maximum(jnp.dot(h, w2_ref[...],
                            preferred_element_type=jnp.float32)
                    + b2_ref[...], 0.0)
    o_ref[...] = jnp.dot(h, wo_ref[...],
                         preferred_element_type=jnp.float32) + bo_ref[...]


def _mlp(pooled, Wg, bg, Wf1, bf1, Wf2, bf2, Wo, bo):
    G = pooled.shape[0]
    Wo_pad = jnp.zeros((16, 128), jnp.float32).at[:, 0].set(Wo[:, 0])
    bo_pad = jnp.zeros((1, 128), jnp.float32).at[0, 0].set(bo[0])
    out = pl.pallas_call(
        _mlp_kernel,
        out_shape=jax.ShapeDtypeStruct((G, 128), jnp.float32),
    )(pooled, Wg, bg.reshape(1, -1), Wf1, bf1.reshape(1, -1),
      Wf2, bf2.reshape(1, -1), Wo_pad, bo_pad)
    return out[:, :1]


NPF = NP * 4


def _p1_body(svs, svd, sd2, cvec, ee_all, den_out,
             svs_v, svd_v, den_v, pk_b, ee_b, cv_b, *, heads):
    c = lax.axis_index("c")
    s = lax.axis_index("s")
    tid = c * NS + s

    pltpu.sync_copy(svs, svs_v)
    pltpu.sync_copy(svd, svd_v)
    pltpu.sync_copy(cvec, cv_b)
    cv = cv_b[...]

    def zrow(i, _):
        den_v[pl.ds(16 * i, 16)] = jnp.zeros((16,), _f32)
        return 0
    lax.fori_loop(0, NPF // 16, zrow, 0)

    def step(st, _):
        gbase = (tid * U1 + st) * K
        pltpu.sync_copy(sd2.at[:, pl.ds(gbase, K)], pk_b)
        for g in range(K // 16):
            sl = pl.ds(16 * g, 16)
            i_s4 = pk_b[0, sl] * 4
            i_d4 = pk_b[1, sl] * 4
            for h in range(heads):
                ss = plsc.load_gather(svs_v, [i_s4 + h])
                sd = plsc.load_gather(svd_v, [i_d4 + h])
                t = ss + sd
                e = jnp.where(t >= 0.0, t, 0.2 * t)
                ee = jnp.exp(e - cv)
                ee_b[h, sl] = ee
                plsc.addupdate_scatter(den_v, [i_d4 + h], ee)
        pltpu.sync_copy(ee_b, ee_all.at[:, pl.ds(gbase, K)])
        return 0
    lax.fori_loop(0, U1, step, 0)
    pltpu.sync_copy(den_v, den_out.at[tid])


def _make_p1(heads):
    return pl.kernel(
        functools.partial(_p1_body, heads=heads),
        out_type=[jax.ShapeDtypeStruct((4, EP), _f32),
                  jax.ShapeDtypeStruct((NC * NS, NPF), _f32)],
        mesh=_MESH,
        scratch_types=[
            pltpu.VMEM((NPF,), _f32), pltpu.VMEM((NPF,), _f32),
            pltpu.VMEM((NPF,), _f32),
            pltpu.VMEM((2, K), _i32), pltpu.VMEM((4, K), _f32),
            pltpu.VMEM((16,), _f32),
        ],
        compiler_params=pltpu.CompilerParams(needs_layout_passes=False),
    )


_p1_h3 = _make_p1(3)
_p1_h1 = _make_p1(1)


def _p4_body(*refs, n_j, steps, split_cores):
    h_all = refs[0]
    pks = refs[1:1 + n_j]
    agg_out = refs[1 + n_j]
    scr = refs[2 + n_j:]
    pkb = scr[0:2]
    idx_a = scr[2:4]
    idx_d3 = scr[4]
    rows = scr[6:8]
    gsem = scr[8:10]
    ssem = scr[10:12]
    acc = scr[12]
    c = lax.axis_index("c")
    s = lax.axis_index("s")

    def gb(st):
        if split_cores:
            return ((c * NS + s) * steps + st) * K
        return (s * steps + st) * K

    for j in range(n_j):
        chunk = 2 * j + c if n_j > 1 else c * 0
        pk = pks[j]

        def zrow(i, _):
            for t in range(8):
                rows[0][i, pl.ds(16 * t, 16)] = jnp.zeros((16,), _f32)
            return 0
        lax.fori_loop(0, K, zrow, 0)
        for r in range(4):
            pltpu.sync_copy(rows[0], acc.at[pl.ds(s * ZR + r * K, K)])
        pltpu.sync_copy(rows[0].at[pl.ds(0, ZR - 4 * K)],
                        acc.at[pl.ds(s * ZR + 4 * K, ZR - 4 * K)])
        plsc.subcore_barrier()

        def prep(b, st, qn):
            pltpu.sync_copy(pk.at[:, pl.ds(gb(st), K)], pkb[b])
            for g16 in range(8):
                sl = pl.ds(16 * g16, 16)
                idx_a[b][sl] = pkb[b][0, sl] + chunk
                idx_d3[qn, sl] = pkb[b][1, sl]

        def gather(b):
            pltpu.async_copy(h_all.at[idx_a[b]], rows[b], gsem[b])

        def wait_g(b):
            pltpu.make_async_copy(h_all.at[idx_a[b]], rows[b],
                                  gsem[b]).wait()

        def scatter(b, q):
            pltpu.async_copy(rows[b], acc.at[idx_d3.at[q]], ssem[b],
                             add=True)

        def wait_s(b):
            pltpu.make_async_copy(rows[b], acc.at[idx_d3.at[0]],
                                  ssem[b]).wait()

        def compute(b):
            def wg(g16, _):
                wv = plsc.bitcast(pkb[b][2, pl.ds(16 * g16, 16)], _f32)
                for lane in range(16):
                    w = wv[lane]
                    k = g16 * 16 + lane
                    for t in range(8):
                        sl = pl.ds(16 * t, 16)
                        rows[b][k, sl] = rows[b][k, sl] * w
                return 0
            lax.fori_loop(0, 8, wg, 0)

        prep(0, 0, 0)
        gather(0)
        prep(1, 1, 1)

        def body2(i, _):
            q0 = lax.rem(2 * i, 4)
            wait_g(0)

            @pl.when(i > 0)
            def _():
                wait_s(1)
            gather(1)
            compute(0)
            scatter(0, q0)

            @pl.when(i < steps // 2 - 1)
            def _():
                prep(0, 2 * i + 2, lax.rem(2 * i + 2, 4))
            wait_g(1)
            wait_s(0)

            @pl.when(i < steps // 2 - 1)
            def _():
                gather(0)
            compute(1)
            scatter(1, lax.rem(2 * i + 1, 4))

            @pl.when(i < steps // 2 - 1)
            def _():
                prep(1, 2 * i + 3, lax.rem(2 * i + 3, 4))
            return 0
        lax.fori_loop(0, steps // 2, body2, 0)
        wait_s(1)
        plsc.subcore_barrier()
        if n_j > 1:
            pltpu.sync_copy(acc.at[pl.ds(s * ZR, ZR)],
                            agg_out.at[chunk, pl.ds(s * ZR, ZR)])
        else:
            pltpu.sync_copy(acc.at[pl.ds(s * ZR, ZR)],
                            agg_out.at[c, pl.ds(s * ZR, ZR)])


def _make_p4(n_j, steps, split_cores, out_shape):
    return pl.kernel(
        functools.partial(_p4_body, n_j=n_j, steps=steps,
                          split_cores=split_cores),
        out_type=jax.ShapeDtypeStruct(out_shape, _f32),
        mesh=_MESH,
        scratch_types=[
            pltpu.VMEM((3, K), _i32), pltpu.VMEM((3, K), _i32),
            pltpu.VMEM((K,), _i32), pltpu.VMEM((K,), _i32),
            pltpu.VMEM((4, K), _i32), pltpu.VMEM((K,), _i32),
            pltpu.VMEM((K, 128), _f32), pltpu.VMEM((K, 128), _f32),
            pltpu.SemaphoreType.DMA, pltpu.SemaphoreType.DMA,
            pltpu.SemaphoreType.DMA, pltpu.SemaphoreType.DMA,
            pltpu.VMEM_SHARED((NP, 128), _f32),
        ],
        compiler_params=pltpu.CompilerParams(needs_layout_passes=False),
    )


_p4_l1 = _make_p4(3, U4, False, (6, NP, 128))
_p4_l2 = _make_p4(1, U1, True, (NC, NP, 128))


def _edge_layer(h_all, s_scores, heads, srcp, dstp, p1, p4):
    s_src = s_scores[:, :heads]
    s_dst = s_scores[:, heads:2 * heads]
    C = jnp.max(s_src) + jnp.max(s_dst)
    cvec = jnp.full((16,), C, _f32)
    svs = jnp.pad(s_src, ((0, 0), (0, 4 - heads))).reshape(NPF)
    svd = jnp.pad(s_dst, ((0, 0), (0, 4 - heads))).reshape(NPF)
    sd2 = jnp.stack([srcp, dstp])
    ee_all, den_part = p1(svs, svd, sd2, cvec)
    den = den_part.sum(axis=0).reshape(NP, 4)
    mult = 6 if heads > 1 else 1
    src_m = srcp * mult
    pks = [jnp.stack([src_m, dstp,
                      lax.bitcast_convert_type(ee_all[h], _i32)])
           for h in range(heads)]
    agg = p4(h_all, *pks)
    return agg, den


def kernel(x, edge_index, batch, W1, a_src1, a_dst1, b1, W2, a_src2,
           a_dst2, b2, Wg, bg, Wf1, bf1, Wf2, bf2, Wo, bo):
    G = 256
    H1 = a_src1.shape[0]
    loop = jnp.arange(N, dtype=edge_index.dtype)
    padi = jnp.full((EP - ETOT,), N, edge_index.dtype)
    srcp = jnp.concatenate([edge_index[0], loop, padi])
    dstp = jnp.concatenate([edge_index[1], loop, padi])

    a2_1 = jnp.stack([a_src1.reshape(-1), a_dst1.reshape(-1)])
    xp = jnp.pad(x, ((0, NP - N), (0, 0)))
    h1, s1 = _matmul_att(xp, W1, a2_1, 632, H1, x.shape[1])
    h_all1 = h1.reshape(NP * 6, 128)
    agg1, den4_1 = _edge_layer(h_all1, s1, H1, srcp, dstp, _p1_h3, _p4_l1)

    a2_2 = jnp.stack([a_src2.reshape(-1), a_dst2.reshape(-1)])
    den128 = jnp.pad(den4_1, ((0, 0), (0, 124)))
    h2, s2 = _l2_fused(agg1, den128, b1, W2, a2_2)
    agg2, den4_2 = _edge_layer(h2, s2, 1, srcp, dstp, _p1_h1, _p4_l2)
    den2s = den4_2[:N, :1]
    x2 = jax.nn.relu((agg2[0] + agg2[1])[:N] / (den2s + 1e-16) + b2)

    pooled = jax.ops.segment_max(x2, batch, num_segments=G)
    pooled = jnp.where(jnp.isfinite(pooled), pooled, 0.0)
    return _mlp(pooled, Wg, bg, Wf1, bf1, Wf2, bf2, Wo, bo)


# speedup vs baseline: 23.1666x; 1.0255x over previous
import functools

import jax
import jax.numpy as jnp
from jax import lax
from jax.experimental import pallas as pl
from jax.experimental.pallas import tpu as pltpu
from jax.experimental.pallas import tpu_sc as plsc

N = 10000
NP = 10112
ZR = NP // 16
E = 160000
ETOT = E + N
K = 128
EP = 172032
NC, NS = 2, 16
U1 = EP // (NC * NS * K)
U4 = EP // (NS * K)

_f32 = jnp.float32
_i32 = jnp.int32

_MESH = plsc.VectorSubcoreMesh(core_axis_name="c", subcore_axis_name="s")


def _mm_att_kernel(x_ref, w_ref, a_ref, h_ref, s_ref, *, heads, out_ch):
    h = jnp.dot(x_ref[...], w_ref[...], preferred_element_type=jnp.float32)
    h_ref[...] = h
    cols = []
    for k in range(2):
        for hd in range(heads):
            sl = slice(hd * out_ch, (hd + 1) * out_ch)
            prod = h[:, sl] * a_ref[k, sl][None, :]
            cols.append(jnp.sum(prod, axis=1, keepdims=True))
    s = jnp.concatenate(cols, axis=1)
    s_ref[...] = jnp.pad(s, ((0, 0), (0, 128 - 2 * heads)))


def _matmul_att(x, w, a2, block_rows, heads, out_ch):
    n, k = x.shape
    _, m = w.shape
    grid = n // block_rows
    return pl.pallas_call(
        functools.partial(_mm_att_kernel, heads=heads, out_ch=out_ch),
        grid=(grid,),
        in_specs=[
            pl.BlockSpec((block_rows, k), lambda i: (i, 0)),
            pl.BlockSpec((k, m), lambda i: (0, 0)),
            pl.BlockSpec((2, m), lambda i: (0, 0)),
        ],
        out_specs=[
            pl.BlockSpec((block_rows, m), lambda i: (i, 0)),
            pl.BlockSpec((block_rows, 128), lambda i: (i, 0)),
        ],
        out_shape=[
            jax.ShapeDtypeStruct((n, m), jnp.float32),
            jax.ShapeDtypeStruct((n, 128), jnp.float32),
        ],
    )(x, w, a2)


def _l2_kernel(a_ref, den_ref, b1_ref, w_ref, a2_ref, h_ref, s_ref):
    i = pl.program_id(0)
    rid = i * 632 + lax.broadcasted_iota(jnp.int32, (632, 1), 0)
    valid = rid < N
    h2 = jnp.zeros((632, 128), jnp.float32)
    for c in range(6):
        dh = den_ref[:, c // 2:c // 2 + 1]
        xc = a_ref[c] / (dh + 1e-16) + b1_ref[0, 128 * c:128 * (c + 1)]
        xc = jnp.where(xc > 0, xc, jnp.exp(jnp.minimum(xc, 0.0)) - 1.0)
        xc = jnp.where(valid, xc, 0.0)
        h2 = h2 + jnp.dot(xc, w_ref[128 * c:128 * (c + 1), :],
                          preferred_element_type=jnp.float32)
    h_ref[...] = h2
    ss = jnp.sum(h2 * a2_ref[0][None, :], axis=1, keepdims=True)
    sd = jnp.sum(h2 * a2_ref[1][None, :], axis=1, keepdims=True)
    s_ref[...] = jnp.pad(jnp.concatenate([ss, sd], axis=1),
                         ((0, 0), (0, 126)))


def _l2_fused(agg1, den128, b1, W2, a2_2):
    grid = NP // 632
    return pl.pallas_call(
        _l2_kernel,
        grid=(grid,),
        in_specs=[
            pl.BlockSpec((6, 632, 128), lambda i: (0, i, 0)),
            pl.BlockSpec((632, 128), lambda i: (i, 0)),
            pl.BlockSpec((1, 768), lambda i: (0, 0)),
            pl.BlockSpec((768, 128), lambda i: (0, 0)),
            pl.BlockSpec((2, 128), lambda i: (0, 0)),
        ],
        out_specs=[
            pl.BlockSpec((632, 128), lambda i: (i, 0)),
            pl.BlockSpec((632, 128), lambda i: (i, 0)),
        ],
        out_shape=[
            jax.ShapeDtypeStruct((NP, 128), jnp.float32),
            jax.ShapeDtypeStruct((NP, 128), jnp.float32),
        ],
    )(agg1, den128, b1.reshape(1, -1), W2, a2_2)


def _mlp_kernel(p_ref, wg_ref, bg_ref, w1_ref, b1_ref, w2_ref, b2_ref,
                wo_ref, bo_ref, o_ref):
    h = jnp.maximum(jnp.dot(p_ref[...], wg_ref[...],
                            preferred_element_type=jnp.float32)
                    + bg_ref[...], 0.0)
    h = jnp.maximum(jnp.dot(h, w1_ref[...],
                            preferred_element_type=jnp.float32)
                    + b1_ref[...], 0.0)
    h = jnp.maximum(jnp.dot(h, w2_ref[...],
                            preferred_element_type=jnp.float32)
                    + b2_ref[...], 0.0)
    o_ref[...] = jnp.dot(h, wo_ref[...],
                         preferred_element_type=jnp.float32) + bo_ref[...]


def _mlp(pooled, Wg, bg, Wf1, bf1, Wf2, bf2, Wo, bo):
    G = pooled.shape[0]
    Wo_pad = jnp.zeros((16, 128), jnp.float32).at[:, 0].set(Wo[:, 0])
    bo_pad = jnp.zeros((1, 128), jnp.float32).at[0, 0].set(bo[0])
    out = pl.pallas_call(
        _mlp_kernel,
        out_shape=jax.ShapeDtypeStruct((G, 128), jnp.float32),
    )(pooled, Wg, bg.reshape(1, -1), Wf1, bf1.reshape(1, -1),
      Wf2, bf2.reshape(1, -1), Wo_pad, bo_pad)
    return out[:, :1]


NPF = NP * 4


def _p1_body(svs, svd, sd2, cvec, ee_all, den_out,
             svs_v, svd_v, den_v, pk0, pk1, ee0, ee1, cv_b,
             lsem0, lsem1, esem0, esem1, *, heads):
    pk_b = (pk0, pk1)
    ee_b = (ee0, ee1)
    lsem = (lsem0, lsem1)
    esem = (esem0, esem1)
    c = lax.axis_index("c")
    s = lax.axis_index("s")
    tid = c * NS + s

    pltpu.sync_copy(svs, svs_v)
    pltpu.sync_copy(svd, svd_v)
    pltpu.sync_copy(cvec, cv_b)
    cv = cv_b[...]

    def zrow(i, _):
        den_v[pl.ds(16 * i, 16)] = jnp.zeros((16,), _f32)
        return 0
    lax.fori_loop(0, NPF // 16, zrow, 0)

    def gb(st):
        return (tid * U1 + st) * K

    def load(b, st):
        pltpu.async_copy(sd2.at[:, pl.ds(gb(st), K)], pk_b[b], lsem[b])

    def wait_l(b, st):
        pltpu.make_async_copy(sd2.at[:, pl.ds(gb(st), K)], pk_b[b],
                              lsem[b]).wait()

    def store(b, st):
        pltpu.async_copy(ee_b[b], ee_all.at[:, pl.ds(gb(st), K)], esem[b])

    def wait_e(b, st):
        pltpu.make_async_copy(ee_b[b], ee_all.at[:, pl.ds(gb(st), K)],
                              esem[b]).wait()

    def compute(b):
        for g in range(K // 16):
            sl = pl.ds(16 * g, 16)
            i_s4 = pk_b[b][0, sl] * 4
            i_d4 = pk_b[b][1, sl] * 4
            for h in range(heads):
                ss = plsc.load_gather(svs_v, [i_s4 + h])
                sd = plsc.load_gather(svd_v, [i_d4 + h])
                t = ss + sd
                e = jnp.where(t >= 0.0, t, 0.2 * t)
                ee = jnp.exp(e - cv)
                ee_b[b][h, sl] = ee
                plsc.addupdate_scatter(den_v, [i_d4 + h], ee)

    load(0, 0)
    load(1, 1)

    def body2(i, _):
        g0 = 2 * i
        wait_l(0, g0)

        @pl.when(i > 0)
        def _():
            wait_e(0, g0 - 2)
        compute(0)
        store(0, g0)

        @pl.when(i < U1 // 2 - 1)
        def _():
            load(0, g0 + 2)
        wait_l(1, g0 + 1)

        @pl.when(i > 0)
        def _():
            wait_e(1, g0 - 1)
        compute(1)
        store(1, g0 + 1)

        @pl.when(i < U1 // 2 - 1)
        def _():
            load(1, g0 + 3)
        return 0
    lax.fori_loop(0, U1 // 2, body2, 0)
    wait_e(0, U1 - 2)
    wait_e(1, U1 - 1)
    pltpu.sync_copy(den_v, den_out.at[tid])


def _make_p1(heads):
    return pl.kernel(
        functools.partial(_p1_body, heads=heads),
        out_type=[jax.ShapeDtypeStruct((4, EP), _f32),
                  jax.ShapeDtypeStruct((NC * NS, NPF), _f32)],
        mesh=_MESH,
        scratch_types=[
            pltpu.VMEM((NPF,), _f32), pltpu.VMEM((NPF,), _f32),
            pltpu.VMEM((NPF,), _f32),
            pltpu.VMEM((2, K), _i32), pltpu.VMEM((2, K), _i32),
            pltpu.VMEM((4, K), _f32), pltpu.VMEM((4, K), _f32),
            pltpu.VMEM((16,), _f32),
            pltpu.SemaphoreType.DMA, pltpu.SemaphoreType.DMA,
            pltpu.SemaphoreType.DMA, pltpu.SemaphoreType.DMA,
        ],
        compiler_params=pltpu.CompilerParams(needs_layout_passes=False),
    )


_p1_h3 = _make_p1(3)
_p1_h1 = _make_p1(1)


def _p4_body(*refs, n_j, steps, split_cores):
    h_all = refs[0]
    pks = refs[1:1 + n_j]
    agg_out = refs[1 + n_j]
    scr = refs[2 + n_j:]
    pkb = scr[0:2]
    idx_a = scr[2:4]
    idx_d3 = scr[4]
    rows = scr[6:8]
    gsem = scr[8:10]
    ssem = scr[10:12]
    acc = scr[12]
    c = lax.axis_index("c")
    s = lax.axis_index("s")

    def gb(st):
        if split_cores:
            return ((c * NS + s) * steps + st) * K
        return (s * steps + st) * K

    for j in range(n_j):
        chunk = 2 * j + c if n_j > 1 else c * 0
        pk = pks[j]

        def zrow(i, _):
            for t in range(8):
                rows[0][i, pl.ds(16 * t, 16)] = jnp.zeros((16,), _f32)
            return 0
        lax.fori_loop(0, K, zrow, 0)
        for r in range(4):
            pltpu.sync_copy(rows[0], acc.at[pl.ds(s * ZR + r * K, K)])
        pltpu.sync_copy(rows[0].at[pl.ds(0, ZR - 4 * K)],
                        acc.at[pl.ds(s * ZR + 4 * K, ZR - 4 * K)])
        plsc.subcore_barrier()

        def prep(b, st, qn):
            pltpu.sync_copy(pk.at[:, pl.ds(gb(st), K)], pkb[b])
            for g16 in range(8):
                sl = pl.ds(16 * g16, 16)
                idx_a[b][sl] = pkb[b][0, sl] + chunk
                idx_d3[qn, sl] = pkb[b][1, sl]

        def gather(b):
            pltpu.async_copy(h_all.at[idx_a[b]], rows[b], gsem[b])

        def wait_g(b):
            pltpu.make_async_copy(h_all.at[idx_a[b]], rows[b],
                                  gsem[b]).wait()

        def scatter(b, q):
            pltpu.async_copy(rows[b], acc.at[idx_d3.at[q]], ssem[b],
                             add=True)

        def wait_s(b):
            pltpu.make_async_copy(rows[b], acc.at[idx_d3.at[0]],
                                  ssem[b]).wait()

        def compute(b):
            def wg(g16, _):
                wv = plsc.bitcast(pkb[b][2, pl.ds(16 * g16, 16)], _f32)
                for lane in range(16):
                    w = wv[lane]
                    k = g16 * 16 + lane
                    for t in range(8):
                        sl = pl.ds(16 * t, 16)
                        rows[b][k, sl] = rows[b][k, sl] * w
                return 0
            lax.fori_loop(0, 8, wg, 0)

        prep(0, 0, 0)
        gather(0)
        prep(1, 1, 1)

        def body2(i, _):
            q0 = lax.rem(2 * i, 4)
            wait_g(0)

            @pl.when(i > 0)
            def _():
                wait_s(1)
            gather(1)
            compute(0)
            scatter(0, q0)

            @pl.when(i < steps // 2 - 1)
            def _():
                prep(0, 2 * i + 2, lax.rem(2 * i + 2, 4))
            wait_g(1)
            wait_s(0)

            @pl.when(i < steps // 2 - 1)
            def _():
                gather(0)
            compute(1)
            scatter(1, lax.rem(2 * i + 1, 4))

            @pl.when(i < steps // 2 - 1)
            def _():
                prep(1, 2 * i + 3, lax.rem(2 * i + 3, 4))
            return 0
        lax.fori_loop(0, steps // 2, body2, 0)
        wait_s(1)
        plsc.subcore_barrier()
        if n_j > 1:
            pltpu.sync_copy(acc.at[pl.ds(s * ZR, ZR)],
                            agg_out.at[chunk, pl.ds(s * ZR, ZR)])
        else:
            pltpu.sync_copy(acc.at[pl.ds(s * ZR, ZR)],
                            agg_out.at[c, pl.ds(s * ZR, ZR)])


def _make_p4(n_j, steps, split_cores, out_shape):
    return pl.kernel(
        functools.partial(_p4_body, n_j=n_j, steps=steps,
                          split_cores=split_cores),
        out_type=jax.ShapeDtypeStruct(out_shape, _f32),
        mesh=_MESH,
        scratch_types=[
            pltpu.VMEM((3, K), _i32), pltpu.VMEM((3, K), _i32),
            pltpu.VMEM((K,), _i32), pltpu.VMEM((K,), _i32),
            pltpu.VMEM((4, K), _i32), pltpu.VMEM((K,), _i32),
            pltpu.VMEM((K, 128), _f32), pltpu.VMEM((K, 128), _f32),
            pltpu.SemaphoreType.DMA, pltpu.SemaphoreType.DMA,
            pltpu.SemaphoreType.DMA, pltpu.SemaphoreType.DMA,
            pltpu.VMEM_SHARED((NP, 128), _f32),
        ],
        compiler_params=pltpu.CompilerParams(needs_layout_passes=False),
    )


_p4_l1 = _make_p4(3, U4, False, (6, NP, 128))
_p4_l2 = _make_p4(1, U1, True, (NC, NP, 128))


def _edge_layer(h_all, s_scores, heads, srcp, dstp, p1, p4):
    s_src = s_scores[:, :heads]
    s_dst = s_scores[:, heads:2 * heads]
    C = jnp.max(s_src) + jnp.max(s_dst)
    cvec = jnp.full((16,), C, _f32)
    svs = jnp.pad(s_src, ((0, 0), (0, 4 - heads))).reshape(NPF)
    svd = jnp.pad(s_dst, ((0, 0), (0, 4 - heads))).reshape(NPF)
    sd2 = jnp.stack([srcp, dstp])
    ee_all, den_part = p1(svs, svd, sd2, cvec)
    den = den_part.sum(axis=0).reshape(NP, 4)
    mult = 6 if heads > 1 else 1
    src_m = srcp * mult
    pks = [jnp.stack([src_m, dstp,
                      lax.bitcast_convert_type(ee_all[h], _i32)])
           for h in range(heads)]
    agg = p4(h_all, *pks)
    return agg, den


def kernel(x, edge_index, batch, W1, a_src1, a_dst1, b1, W2, a_src2,
           a_dst2, b2, Wg, bg, Wf1, bf1, Wf2, bf2, Wo, bo):
    G = 256
    H1 = a_src1.shape[0]
    loop = jnp.arange(N, dtype=edge_index.dtype)
    padi = jnp.full((EP - ETOT,), N, edge_index.dtype)
    srcp = jnp.concatenate([edge_index[0], loop, padi])
    dstp = jnp.concatenate([edge_index[1], loop, padi])

    a2_1 = jnp.stack([a_src1.reshape(-1), a_dst1.reshape(-1)])
    xp = jnp.pad(x, ((0, NP - N), (0, 0)))
    h1, s1 = _matmul_att(xp, W1, a2_1, 632, H1, x.shape[1])
    h_all1 = h1.reshape(NP * 6, 128)
    agg1, den4_1 = _edge_layer(h_all1, s1, H1, srcp, dstp, _p1_h3, _p4_l1)

    a2_2 = jnp.stack([a_src2.reshape(-1), a_dst2.reshape(-1)])
    den128 = jnp.pad(den4_1, ((0, 0), (0, 124)))
    h2, s2 = _l2_fused(agg1, den128, b1, W2, a2_2)
    agg2, den4_2 = _edge_layer(h2, s2, 1, srcp, dstp, _p1_h1, _p4_l2)
    den2s = den4_2[:N, :1]
    x2 = jax.nn.relu((agg2[0] + agg2[1])[:N] / (den2s + 1e-16) + b2)

    pooled = jax.ops.segment_max(x2, batch, num_segments=G)
    pooled = jnp.where(jnp.isfinite(pooled), pooled, 0.0)
    return _mlp(pooled, Wg, bg, Wf1, bf1, Wf2, bf2, Wo, bo)
